# Initial kernel scaffold; baseline (speedup 1.0000x reference)
import jax
import jax.numpy as jnp
from jax.experimental import pallas as pl


def kernel(x_atom, x_defect, edge_index_aa, edge_index_ad, edge_index_da, edge_attr_aa, edge_attr_ad, edge_attr_da, state, batch_atom, batch_defect, bond_batch_aa, bond_batch_ad, bond_batch_da, params):
    raise NotImplementedError("write your pallas kernel here")


# speedup vs baseline: 1.7661x; 1.7661x over previous
import functools
import math

import jax
import jax.numpy as jnp
from jax.experimental import pallas as pl
from jax.experimental.pallas import tpu as pltpu

_LN2 = 0.6931471805599453
_EMB = 32
_B = 64


def _ssp(x):
    m = jnp.maximum(x, 0.0)
    return m + jnp.log(jnp.exp(x - m) + jnp.exp(-m)) - _LN2


def _mm(a, b):
    return jax.lax.dot_general(a, b, (((1,), (0,)), ((), ())),
                               preferred_element_type=jnp.float32)


def _mmT0(a, b):
    return jax.lax.dot_general(a, b, (((0,), (0,)), ((), ())),
                               preferred_element_type=jnp.float32)


def _rowdot(a, v):
    return jnp.sum(a * v, axis=1, keepdims=True)


def _t_row(v):
    eye = (jax.lax.broadcasted_iota(jnp.int32, (_B, _B), 0) ==
           jax.lax.broadcasted_iota(jnp.int32, (_B, _B), 1)).astype(jnp.float32)
    return jnp.sum(eye * v, axis=1, keepdims=True)


def _blk(n, cap):
    for d in range(min(n, cap), 0, -1):
        if n % d == 0:
            return d
    return 1


def _onehot(idx_col, nseg):
    cols = jax.lax.broadcasted_iota(jnp.int32, (idx_col.shape[0], nseg), 1)
    return (idx_col == cols).astype(jnp.float32)


def _pre_nodes_body(x_ref, w_ref, b_ref, o_ref):
    o_ref[...] = _mm(x_ref[...], w_ref[0]) + b_ref[0]


def _pre_nodes(xcat, w2, b2, n_atom, n_defect):
    din = xcat.shape[1]
    nb = _blk(math.gcd(n_atom, n_defect), 1000)
    nba = n_atom // nb
    grid = (n_atom + n_defect) // nb

    def nt(i):
        return jnp.where(i >= nba, 1, 0)

    return pl.pallas_call(
        _pre_nodes_body,
        grid=(grid,),
        in_specs=[
            pl.BlockSpec((nb, din), lambda i: (i, 0)),
            pl.BlockSpec((1, din, _EMB), lambda i: (nt(i), 0, 0)),
            pl.BlockSpec((1, 1, _EMB), lambda i: (nt(i), 0, 0)),
        ],
        out_specs=pl.BlockSpec((nb, _EMB), lambda i: (i, 0)),
        out_shape=jax.ShapeDtypeStruct((n_atom + n_defect, _EMB), jnp.float32),
    )(xcat, w2, b2)


def _lin_body(x_ref, w_ref, b_ref, o_ref):
    o_ref[...] = _mm(x_ref[...], w_ref[...]) + b_ref[...]


def _lin_small(x, w, b2):
    return pl.pallas_call(
        _lin_body,
        out_shape=jax.ShapeDtypeStruct((x.shape[0], w.shape[1]), jnp.float32),
    )(x, w, b2)


def _edge_body(inner_skip, ea_ref, gs_ref, gd_ref, bb_ref, up_ref,
               wpre_ref, bpre_ref, w1_ref, b1_ref, w2_ref, b2_ref,
               ne_ref, eanew_ref, es_ref):
    i = pl.program_id(0)
    ep = _mm(ea_ref[...], wpre_ref[0]) + bpre_ref[0]
    oh = _onehot(bb_ref[...], _B)
    ub = _mm(oh, up_ref[...])
    feat = jnp.concatenate([gs_ref[...], gd_ref[...], ep, ub], axis=1)
    h = _ssp(_mm(feat, w1_ref[0]) + b1_ref[0])
    ne = _ssp(_mm(h, w2_ref[0]) + b2_ref[0])
    ne_ref[...] = ne
    skip = ep if inner_skip else ea_ref[...]
    eanew_ref[...] = ne + skip

    @pl.when(i == 0)
    def _():
        es_ref[...] = jnp.zeros_like(es_ref)

    es_ref[...] += _mmT0(oh, ne)


def _edge_layer(eacat, gs, gd, bb2d, up, wpre, bpre, w1, b1, w2, b2,
                inner_skip, nb_aa, nb_ad, eblk):
    n_e = eacat.shape[0]
    din = eacat.shape[1]
    grid = n_e // eblk

    def et(i):
        return jnp.where(i >= nb_aa, 1, 0) + jnp.where(i >= nb_aa + nb_ad, 1, 0)

    return pl.pallas_call(
        functools.partial(_edge_body, inner_skip),
        grid=(grid,),
        in_specs=[
            pl.BlockSpec((eblk, din), lambda i: (i, 0)),
            pl.BlockSpec((eblk, _EMB), lambda i: (i, 0)),
            pl.BlockSpec((eblk, _EMB), lambda i: (i, 0)),
            pl.BlockSpec((eblk, 1), lambda i: (i, 0)),
            pl.BlockSpec((_B, _EMB), lambda i: (0, 0)),
            pl.BlockSpec((1, din, _EMB), lambda i: (et(i), 0, 0)),
            pl.BlockSpec((1, 1, _EMB), lambda i: (et(i), 0, 0)),
            pl.BlockSpec((1, 4 * _EMB, 2 * _EMB), lambda i: (et(i), 0, 0)),
            pl.BlockSpec((1, 1, 2 * _EMB), lambda i: (et(i), 0, 0)),
            pl.BlockSpec((1, 2 * _EMB, _EMB), lambda i: (et(i), 0, 0)),
            pl.BlockSpec((1, 1, _EMB), lambda i: (et(i), 0, 0)),
        ],
        out_specs=[
            pl.BlockSpec((eblk, _EMB), lambda i: (i, 0)),
            pl.BlockSpec((eblk, _EMB), lambda i: (i, 0)),
            pl.BlockSpec((_B, _EMB), lambda i: (0, 0)),
        ],
        out_shape=[
            jax.ShapeDtypeStruct((n_e, _EMB), jnp.float32),
            jax.ShapeDtypeStruct((n_e, _EMB), jnp.float32),
            jax.ShapeDtypeStruct((_B, _EMB), jnp.float32),
        ],
    )(eacat, gs, gd, bb2d, up, wpre, bpre, w1, b1, w2, b2)


def _phiv_body(xp_ref, tot_ref, cnt_ref, bat_ref, up_ref, skip_ref,
               w1_ref, b1_ref, w2_ref, b2_ref, xn_ref, vs_ref):
    i = pl.program_id(0)
    tot = tot_ref[0] + tot_ref[1]
    agg = tot / jnp.maximum(cnt_ref[...], 1.0)
    oh = _onehot(bat_ref[...], _B)
    ub = _mm(oh, up_ref[...])
    feat = jnp.concatenate([xp_ref[...], agg, ub], axis=1)
    h = _ssp(_mm(feat, w1_ref[0]) + b1_ref[0])
    nx = _ssp(_mm(h, w2_ref[0]) + b2_ref[0])
    xn_ref[...] = nx + skip_ref[...]

    @pl.when(i == 0)
    def _():
        vs_ref[...] = jnp.zeros_like(vs_ref)

    vs_ref[...] += _mmT0(oh, nx)


def _phiv_layer(xpcat, tot2, cntcat, bat2d, up, skipcat,
                w1, b1, w2, b2, n_atom, n_defect):
    nb = _blk(math.gcd(n_atom, n_defect), 1000)
    nba = n_atom // nb
    n_all = n_atom + n_defect
    grid = n_all // nb

    def nt(i):
        return jnp.where(i >= nba, 1, 0)

    return pl.pallas_call(
        _phiv_body,
        grid=(grid,),
        in_specs=[
            pl.BlockSpec((nb, _EMB), lambda i: (i, 0)),
            pl.BlockSpec((2, nb, _EMB), lambda i: (0, i, 0)),
            pl.BlockSpec((nb, 1), lambda i: (i, 0)),
            pl.BlockSpec((nb, 1), lambda i: (i, 0)),
            pl.BlockSpec((_B, _EMB), lambda i: (0, 0)),
            pl.BlockSpec((nb, _EMB), lambda i: (i, 0)),
            pl.BlockSpec((1, 3 * _EMB, 2 * _EMB), lambda i: (nt(i), 0, 0)),
            pl.BlockSpec((1, 1, 2 * _EMB), lambda i: (nt(i), 0, 0)),
            pl.BlockSpec((1, 2 * _EMB, _EMB), lambda i: (nt(i), 0, 0)),
            pl.BlockSpec((1, 1, _EMB), lambda i: (nt(i), 0, 0)),
        ],
        out_specs=[
            pl.BlockSpec((nb, _EMB), lambda i: (i, 0)),
            pl.BlockSpec((_B, _EMB), lambda i: (0, 0)),
        ],
        out_shape=[
            jax.ShapeDtypeStruct((n_all, _EMB), jnp.float32),
            jax.ShapeDtypeStruct((_B, _EMB), jnp.float32),
        ],
    )(xpcat, tot2, cntcat, bat2d, up, skipcat, w1, b1, w2, b2)


def _phiu_body(vs_ref, vc_ref, es_ref, ec_ref, up_ref, su_ref,
               w1_ref, b1_ref, w2_ref, b2_ref, o_ref):
    va = vs_ref[...] / jnp.maximum(vc_ref[...], 1.0)
    eag = es_ref[...] / jnp.maximum(ec_ref[...], 1.0)
    feat = jnp.concatenate([va, eag, up_ref[...]], axis=1)
    h = _ssp(_mm(feat, w1_ref[...]) + b1_ref[...])
    nu = _ssp(_mm(h, w2_ref[...]) + b2_ref[...])
    o_ref[...] = nu + su_ref[...]


def _phiu(vs, vc, es, ec, up, su, w1, b1, w2, b2):
    return pl.pallas_call(
        _phiu_body,
        out_shape=jax.ShapeDtypeStruct((_B, _EMB), jnp.float32),
    )(vs, vc, es, ec, up, su, w1, b1, w2, b2)


def _q_from_bias(bih_ref, bhh_ref):
    gates = bih_ref[...] + bhh_ref[...]
    i_ = gates[:, 0 * _EMB:1 * _EMB]
    f_ = gates[:, 1 * _EMB:2 * _EMB]
    g_ = gates[:, 2 * _EMB:3 * _EMB]
    o_ = gates[:, 3 * _EMB:4 * _EMB]
    c = jax.nn.sigmoid(i_) * jnp.tanh(g_)
    h = jax.nn.sigmoid(o_) * jnp.tanh(c)
    return h


def _s2s_nodes_body(x_ref, bat_ref, bih_ref, bhh_ref, o_ref):
    q = _q_from_bias(bih_ref, bhh_ref)
    x = x_ref[...]
    e = _rowdot(x, q)
    oh = _onehot(bat_ref[...], _B)
    em = jnp.where(oh > 0.0, e, -1e30)
    m = jnp.max(em, axis=0, keepdims=True)
    m = jnp.where(m < -1e29, 0.0, m)
    mpn = _rowdot(oh, m)
    ex = jnp.exp(e - mpn)
    den = jnp.sum(oh * ex, axis=0, keepdims=True)
    denpn = _rowdot(oh, den)
    a = ex / (denpn + 1e-16)
    r = _mmT0(oh, a * x)
    o_ref[...] = jnp.concatenate(
        [jnp.broadcast_to(q, (_B, _EMB)), r], axis=1)


def _s2s_nodes(x, bat2d, bih, bhh):
    return pl.pallas_call(
        _s2s_nodes_body,
        out_shape=jax.ShapeDtypeStruct((_B, 2 * _EMB), jnp.float32),
    )(x, bat2d, bih, bhh)


def _s2s_edges_body(x_ref, bat_ref, bih_ref, bhh_ref, o_ref,
                    m_s, den_s, rn_s):
    i = pl.program_id(0)
    nsteps = pl.num_programs(0)

    @pl.when(i == 0)
    def _():
        m_s[...] = jnp.full_like(m_s, -1e30)
        den_s[...] = jnp.zeros_like(den_s)
        rn_s[...] = jnp.zeros_like(rn_s)

    q = _q_from_bias(bih_ref, bhh_ref)
    x = x_ref[...]
    e = _rowdot(x, q)
    oh = _onehot(bat_ref[...], _B)
    em = jnp.where(oh > 0.0, e, -1e30)
    mb = jnp.max(em, axis=0, keepdims=True)
    m_old = m_s[...]
    m_new = jnp.maximum(m_old, mb)
    scale = jnp.exp(m_old - m_new)
    mpn = _rowdot(oh, m_new)
    ex = jnp.exp(e - mpn)
    den_b = jnp.sum(oh * ex, axis=0, keepdims=True)
    rn_b = _mmT0(oh, ex * x)
    scale_col = _t_row(scale)
    m_s[...] = m_new
    den_s[...] = den_s[...] * scale + den_b
    rn_s[...] = rn_s[...] * scale_col + rn_b

    @pl.when(i == nsteps - 1)
    def _():
        den_col = _t_row(den_s[...])
        r = rn_s[...] / (den_col + 1e-16)
        o_ref[...] = jnp.concatenate(
            [jnp.broadcast_to(q, (_B, _EMB)), r], axis=1)


def _s2s_edges(x, bat2d, bih, bhh, eblk):
    n = x.shape[0]
    return pl.pallas_call(
        _s2s_edges_body,
        grid=(n // eblk,),
        in_specs=[
            pl.BlockSpec((eblk, _EMB), lambda i: (i, 0)),
            pl.BlockSpec((eblk, 1), lambda i: (i, 0)),
            pl.BlockSpec((1, 4 * _EMB), lambda i: (0, 0)),
            pl.BlockSpec((1, 4 * _EMB), lambda i: (0, 0)),
        ],
        out_specs=pl.BlockSpec((_B, 2 * _EMB), lambda i: (0, 0)),
        out_shape=jax.ShapeDtypeStruct((_B, 2 * _EMB), jnp.float32),
        scratch_shapes=[
            pltpu.VMEM((1, _B), jnp.float32),
            pltpu.VMEM((1, _B), jnp.float32),
            pltpu.VMEM((_B, _EMB), jnp.float32),
        ],
    )(x, bat2d, bih, bhh)


def _head_body(xa_ref, xd_ref, es_ref, u_ref, w1_ref, b1_ref,
               w2_ref, b2_ref, w3_ref, b3_ref, o_ref):
    feat = jnp.concatenate(
        [xa_ref[...], xd_ref[...], es_ref[...], u_ref[...]], axis=1)
    h = _ssp(_mm(feat, w1_ref[...]) + b1_ref[...])
    h = _ssp(_mm(h, w2_ref[...]) + b2_ref[...])
    o_ref[...] = _mm(h, w3_ref[...]) + b3_ref[...]


def _head(xa, xd, es, u, w1, b1, w2, b2, w3, b3):
    return pl.pallas_call(
        _head_body,
        out_shape=jax.ShapeDtypeStruct((_B, 1), jnp.float32),
    )(xa, xd, es, u, w1, b1, w2, b2, w3, b3)


def _stack_lin(plist):
    w = jnp.stack([p["W"] for p in plist])
    b = jnp.stack([p["b"][None, :] for p in plist])
    return w, b


def _layer_weights(p):
    nts = ("atom", "defect")
    ets = ("aa", "ad", "da")
    wn, bn = _stack_lin([p["pre_node"][nt] for nt in nts])
    we, be = _stack_lin([p["pre_edge"][et] for et in ets])
    w1e, b1e = _stack_lin([p["phi_e"][et][0] for et in ets])
    w2e, b2e = _stack_lin([p["phi_e"][et][1] for et in ets])
    w1v, b1v = _stack_lin([p["phi_v"][nt][0] for nt in nts])
    w2v, b2v = _stack_lin([p["phi_v"][nt][1] for nt in nts])
    return dict(wn=wn, bn=bn, we=we, be=be, w1e=w1e, b1e=b1e, w2e=w2e,
                b2e=b2e, w1v=w1v, b1v=b1v, w2v=w2v, b2v=b2v,
                ws=p["pre_state"]["W"], bs=p["pre_state"]["b"][None, :],
                w1u=p["phi_u"][0]["W"], b1u=p["phi_u"][0]["b"][None, :],
                w2u=p["phi_u"][1]["W"], b2u=p["phi_u"][1]["b"][None, :])


def kernel(x_atom, x_defect, edge_index_aa, edge_index_ad, edge_index_da,
           edge_attr_aa, edge_attr_ad, edge_attr_da, state,
           batch_atom, batch_defect, bond_batch_aa, bond_batch_ad,
           bond_batch_da, params):
    n_atom = x_atom.shape[0]
    n_defect = x_defect.shape[0]
    n_aa = edge_index_aa.shape[1]
    n_ad = edge_index_ad.shape[1]
    n_da = edge_index_da.shape[1]
    n_all = n_atom + n_defect
    n_e = n_aa + n_ad + n_da
    eblk = _blk(math.gcd(n_aa, n_ad, n_da), 2000)
    nb_aa, nb_ad = n_aa // eblk, n_ad // eblk

    srccat = jnp.concatenate([
        edge_index_aa[0], edge_index_ad[0], edge_index_da[0] + n_atom])
    dstcat = jnp.concatenate([
        edge_index_aa[1], edge_index_ad[1] + n_atom, edge_index_da[1]])
    bbcat = jnp.concatenate([bond_batch_aa, bond_batch_ad, bond_batch_da])
    bb2d = bbcat[:, None]
    batcat = jnp.concatenate([batch_atom, batch_defect])
    bat2d = batcat[:, None]

    ones_e = jnp.ones((n_e,), jnp.float32)
    cntcat = jax.ops.segment_sum(ones_e, dstcat, num_segments=n_all)[:, None]
    vc = jax.ops.segment_sum(jnp.ones((n_all,), jnp.float32), batcat,
                             num_segments=_B)[:, None]
    ec = jax.ops.segment_sum(ones_e, bbcat, num_segments=_B)[:, None]

    xcat = jnp.concatenate([x_atom, x_defect], axis=0)
    eacat = jnp.concatenate([edge_attr_aa, edge_attr_ad, edge_attr_da], axis=0)
    u = state

    for li, pk in enumerate(("m1", "b1", "b2")):
        w = _layer_weights(params[pk])
        inner = (li == 0)
        xp = _pre_nodes(xcat, w["wn"], w["bn"], n_atom, n_defect)
        up = _lin_small(u, w["ws"], w["bs"])
        gs = jnp.take(xp, srccat, axis=0)
        gd = jnp.take(xp, dstcat, axis=0)
        newe, eanew, es = _edge_layer(
            eacat, gs, gd, bb2d, up, w["we"], w["be"], w["w1e"], w["b1e"],
            w["w2e"], w["b2e"], inner, nb_aa, nb_ad, eblk)
        tot = jax.ops.segment_sum(newe, dstcat, num_segments=n_all)
        tot2 = jnp.stack([tot, jnp.zeros_like(tot)])
        skip = xp if inner else xcat
        xnew, vs = _phiv_layer(xp, tot2, cntcat, bat2d, up, skip,
                               w["w1v"], w["b1v"], w["w2v"], w["b2v"],
                               n_atom, n_defect)
        su = up if inner else u
        u = _phiu(vs, vc, es, ec, up, su,
                  w["w1u"], w["b1u"], w["w2u"], w["b2u"])
        xcat = xnew
        eacat = eanew

    pv = params["sv"]
    xa = _s2s_nodes(xcat[:n_atom], bat2d[:n_atom],
                    (pv["b_ih"] + 0.0)[None, :], pv["b_hh"][None, :])
    pv2 = params["sv2"]
    xd = _s2s_nodes(xcat[n_atom:], bat2d[n_atom:],
                    pv2["b_ih"][None, :], pv2["b_hh"][None, :])
    pe = params["se"]
    es2 = _s2s_edges(eacat, bb2d, pe["b_ih"][None, :], pe["b_hh"][None, :],
                     eblk)
    return _head(xa, xd, es2, u,
                 params["h1"]["W"], params["h1"]["b"][None, :],
                 params["h2"]["W"], params["h2"]["b"][None, :],
                 params["h3"]["W"], params["h3"]["b"][None, :])


# speedup vs baseline: 4.3995x; 2.4911x over previous
import functools
import math

import jax
import jax.numpy as jnp
from jax import lax
from jax.experimental import pallas as pl
from jax.experimental.pallas import tpu as pltpu
from jax.experimental.pallas import tpu_sc as plsc

_LN2 = 0.6931471805599453
_EMB = 32
_B = 64


def _ssp(x):
    m = jnp.maximum(x, 0.0)
    return m + jnp.log(jnp.exp(x - m) + jnp.exp(-m)) - _LN2


def _mm(a, b):
    return jax.lax.dot_general(a, b, (((1,), (0,)), ((), ())),
                               preferred_element_type=jnp.float32)


def _mmT0(a, b):
    return jax.lax.dot_general(a, b, (((0,), (0,)), ((), ())),
                               preferred_element_type=jnp.float32)


def _rowdot(a, v):
    return jnp.sum(a * v, axis=1, keepdims=True)


def _t_row(v):
    eye = (jax.lax.broadcasted_iota(jnp.int32, (_B, _B), 0) ==
           jax.lax.broadcasted_iota(jnp.int32, (_B, _B), 1)).astype(jnp.float32)
    return jnp.sum(eye * v, axis=1, keepdims=True)


def _blk(n, cap):
    for d in range(min(n, cap), 0, -1):
        if n % d == 0:
            return d
    return 1


def _onehot(idx_col, nseg):
    cols = jax.lax.broadcasted_iota(jnp.int32, (idx_col.shape[0], nseg), 1)
    return (idx_col == cols).astype(jnp.float32)


_CHUNK = 128


def _sc_gather(table, srccat, dstcat):
    info = plsc.get_sparse_core_info()
    nw = info.num_cores * info.num_subcores
    n_e = srccat.shape[0]
    nchunks = n_e // _CHUNK
    mesh = plsc.VectorSubcoreMesh(core_axis_name="c", subcore_axis_name="s")

    @functools.partial(
        pl.kernel, mesh=mesh,
        out_type=[jax.ShapeDtypeStruct((n_e, 128), jnp.float32),
                  jax.ShapeDtypeStruct((n_e, 128), jnp.float32)],
        scratch_types=[pltpu.VMEM((_CHUNK,), jnp.int32),
                       pltpu.VMEM((_CHUNK, 128), jnp.float32),
                       pltpu.SemaphoreType.DMA],
    )
    def k(table_h, src_h, dst_h, gs_h, gd_h, idx_v, rows_v, sem):
        w = lax.axis_index("s") * info.num_cores + lax.axis_index("c")
        c0 = w * nchunks // nw
        c1 = (w + 1) * nchunks // nw

        def do(idx_h, out_h):
            def body(j, carry):
                b = (c0 + j) * _CHUNK
                pltpu.sync_copy(idx_h.at[pl.ds(b, _CHUNK)], idx_v)
                pltpu.async_copy(table_h.at[idx_v], rows_v, sem).wait()
                pltpu.sync_copy(rows_v, out_h.at[pl.ds(b, _CHUNK)])
                return carry
            lax.fori_loop(0, c1 - c0, body, 0)

        do(src_h, gs_h)
        do(dst_h, gd_h)

    return k(table, srccat, dstcat)


def _sc_scatter(newe, dstcat, zeros_hbm, n_all):
    info = plsc.get_sparse_core_info()
    nc, ns = info.num_cores, info.num_subcores
    n_e = newe.shape[0]
    nchunks = n_e // _CHUNK
    per_core = nchunks // nc
    nzs = 10
    stripe = n_all // nzs
    mesh = plsc.VectorSubcoreMesh(core_axis_name="c", subcore_axis_name="s")

    @functools.partial(
        pl.kernel, mesh=mesh,
        out_type=jax.ShapeDtypeStruct((nc, n_all, 128), jnp.float32),
        scratch_types=[pltpu.VMEM((_CHUNK,), jnp.int32),
                       pltpu.VMEM((_CHUNK, 128), jnp.float32),
                       pltpu.VMEM_SHARED((n_all, 128), jnp.float32),
                       pltpu.SemaphoreType.DMA],
    )
    def k(ne_h, dst_h, z_h, tot_h, idx_v, rows_v, acc_sh, sem):
        c = lax.axis_index("c")
        s = lax.axis_index("s")

        @pl.when(s < nzs)
        def _():
            pltpu.sync_copy(z_h.at[pl.ds(s * stripe, stripe)],
                            acc_sh.at[pl.ds(s * stripe, stripe)])
        plsc.subcore_barrier()

        c0 = c * per_core + s * per_core // ns
        c1 = c * per_core + (s + 1) * per_core // ns

        def body(j, carry):
            b = (c0 + j) * _CHUNK
            pltpu.sync_copy(dst_h.at[pl.ds(b, _CHUNK)], idx_v)
            pltpu.sync_copy(ne_h.at[pl.ds(b, _CHUNK)], rows_v)
            pltpu.sync_copy(rows_v, acc_sh.at[idx_v], add=True)
            return carry
        lax.fori_loop(0, c1 - c0, body, 0)
        plsc.subcore_barrier()

        @pl.when(s < nzs)
        def _():
            pltpu.sync_copy(acc_sh.at[pl.ds(s * stripe, stripe)],
                            tot_h.at[c].at[pl.ds(s * stripe, stripe)])

    return k(newe, dstcat, zeros_hbm)


def _pre_nodes_body(x_ref, w_ref, b_ref, o_ref):
    res = _mm(x_ref[...], w_ref[0]) + b_ref[0]
    o_ref[...] = jnp.concatenate(
        [res, jnp.zeros((res.shape[0], 128 - _EMB), jnp.float32)], axis=1)


def _pre_nodes(xcat, w2, b2, n_atom, n_defect):
    din = xcat.shape[1]
    nb = _blk(math.gcd(n_atom, n_defect), 1000)
    nba = n_atom // nb
    grid = (n_atom + n_defect) // nb

    def nt(i):
        return jnp.where(i >= nba, 1, 0)

    return pl.pallas_call(
        _pre_nodes_body,
        grid=(grid,),
        in_specs=[
            pl.BlockSpec((nb, din), lambda i: (i, 0)),
            pl.BlockSpec((1, din, _EMB), lambda i: (nt(i), 0, 0)),
            pl.BlockSpec((1, 1, _EMB), lambda i: (nt(i), 0, 0)),
        ],
        out_specs=pl.BlockSpec((nb, 128), lambda i: (i, 0)),
        out_shape=jax.ShapeDtypeStruct((n_atom + n_defect, 128), jnp.float32),
    )(xcat, w2, b2)


def _lin_body(x_ref, w_ref, b_ref, o_ref):
    o_ref[...] = _mm(x_ref[...], w_ref[...]) + b_ref[...]


def _lin_small(x, w, b2):
    return pl.pallas_call(
        _lin_body,
        out_shape=jax.ShapeDtypeStruct((x.shape[0], w.shape[1]), jnp.float32),
    )(x, w, b2)


def _edge_body(inner_skip, ea_ref, gs_ref, gd_ref, bb_ref, up_ref,
               wpre_ref, bpre_ref, w1_ref, b1_ref, w2_ref, b2_ref,
               ne_ref, eanew_ref, es_ref):
    i = pl.program_id(0)
    ep = _mm(ea_ref[...], wpre_ref[0]) + bpre_ref[0]
    oh = _onehot(bb_ref[...], _B)
    ub = _mm(oh, up_ref[...])
    gs = gs_ref[...][:, :_EMB]
    gd = gd_ref[...][:, :_EMB]
    feat = jnp.concatenate([gs, gd, ep, ub], axis=1)
    h = _ssp(_mm(feat, w1_ref[0]) + b1_ref[0])
    ne = _ssp(_mm(h, w2_ref[0]) + b2_ref[0])
    ne_ref[...] = jnp.concatenate(
        [ne, jnp.zeros((ne.shape[0], 128 - _EMB), jnp.float32)], axis=1)
    skip = ep if inner_skip else ea_ref[...]
    eanew_ref[...] = ne + skip

    @pl.when(i == 0)
    def _():
        es_ref[...] = jnp.zeros_like(es_ref)

    es_ref[...] += _mmT0(oh, ne)


def _edge_layer(eacat, gs, gd, bb2d, up, wpre, bpre, w1, b1, w2, b2,
                inner_skip, nb_aa, nb_ad, eblk):
    n_e = eacat.shape[0]
    din = eacat.shape[1]
    grid = n_e // eblk

    def et(i):
        return jnp.where(i >= nb_aa, 1, 0) + jnp.where(i >= nb_aa + nb_ad, 1, 0)

    return pl.pallas_call(
        functools.partial(_edge_body, inner_skip),
        grid=(grid,),
        in_specs=[
            pl.BlockSpec((eblk, din), lambda i: (i, 0)),
            pl.BlockSpec((eblk, 128), lambda i: (i, 0)),
            pl.BlockSpec((eblk, 128), lambda i: (i, 0)),
            pl.BlockSpec((eblk, 1), lambda i: (i, 0)),
            pl.BlockSpec((_B, _EMB), lambda i: (0, 0)),
            pl.BlockSpec((1, din, _EMB), lambda i: (et(i), 0, 0)),
            pl.BlockSpec((1, 1, _EMB), lambda i: (et(i), 0, 0)),
            pl.BlockSpec((1, 4 * _EMB, 2 * _EMB), lambda i: (et(i), 0, 0)),
            pl.BlockSpec((1, 1, 2 * _EMB), lambda i: (et(i), 0, 0)),
            pl.BlockSpec((1, 2 * _EMB, _EMB), lambda i: (et(i), 0, 0)),
            pl.BlockSpec((1, 1, _EMB), lambda i: (et(i), 0, 0)),
        ],
        out_specs=[
            pl.BlockSpec((eblk, 128), lambda i: (i, 0)),
            pl.BlockSpec((eblk, _EMB), lambda i: (i, 0)),
            pl.BlockSpec((_B, _EMB), lambda i: (0, 0)),
        ],
        out_shape=[
            jax.ShapeDtypeStruct((n_e, 128), jnp.float32),
            jax.ShapeDtypeStruct((n_e, _EMB), jnp.float32),
            jax.ShapeDtypeStruct((_B, _EMB), jnp.float32),
        ],
    )(eacat, gs, gd, bb2d, up, wpre, bpre, w1, b1, w2, b2)


def _phiv_body(xp_ref, tot_ref, cnt_ref, bat_ref, up_ref, skip_ref,
               w1_ref, b1_ref, w2_ref, b2_ref, xn_ref, vs_ref):
    i = pl.program_id(0)
    xp = xp_ref[...][:, :_EMB]
    tot = tot_ref[0][:, :_EMB] + tot_ref[1][:, :_EMB]
    agg = tot / jnp.maximum(cnt_ref[...], 1.0)
    oh = _onehot(bat_ref[...], _B)
    ub = _mm(oh, up_ref[...])
    feat = jnp.concatenate([xp, agg, ub], axis=1)
    h = _ssp(_mm(feat, w1_ref[0]) + b1_ref[0])
    nx = _ssp(_mm(h, w2_ref[0]) + b2_ref[0])
    xn_ref[...] = nx + skip_ref[...][:, :_EMB]

    @pl.when(i == 0)
    def _():
        vs_ref[...] = jnp.zeros_like(vs_ref)

    vs_ref[...] += _mmT0(oh, nx)


def _phiv_layer(xpcat, tot2, cntcat, bat2d, up, skipcat,
                w1, b1, w2, b2, n_atom, n_defect):
    nb = _blk(math.gcd(n_atom, n_defect), 1000)
    nba = n_atom // nb
    n_all = n_atom + n_defect
    grid = n_all // nb

    def nt(i):
        return jnp.where(i >= nba, 1, 0)

    return pl.pallas_call(
        _phiv_body,
        grid=(grid,),
        in_specs=[
            pl.BlockSpec((nb, xpcat.shape[1]), lambda i: (i, 0)),
            pl.BlockSpec((2, nb, tot2.shape[2]), lambda i: (0, i, 0)),
            pl.BlockSpec((nb, 1), lambda i: (i, 0)),
            pl.BlockSpec((nb, 1), lambda i: (i, 0)),
            pl.BlockSpec((_B, _EMB), lambda i: (0, 0)),
            pl.BlockSpec((nb, skipcat.shape[1]), lambda i: (i, 0)),
            pl.BlockSpec((1, 3 * _EMB, 2 * _EMB), lambda i: (nt(i), 0, 0)),
            pl.BlockSpec((1, 1, 2 * _EMB), lambda i: (nt(i), 0, 0)),
            pl.BlockSpec((1, 2 * _EMB, _EMB), lambda i: (nt(i), 0, 0)),
            pl.BlockSpec((1, 1, _EMB), lambda i: (nt(i), 0, 0)),
        ],
        out_specs=[
            pl.BlockSpec((nb, _EMB), lambda i: (i, 0)),
            pl.BlockSpec((_B, _EMB), lambda i: (0, 0)),
        ],
        out_shape=[
            jax.ShapeDtypeStruct((n_all, _EMB), jnp.float32),
            jax.ShapeDtypeStruct((_B, _EMB), jnp.float32),
        ],
    )(xpcat, tot2, cntcat, bat2d, up, skipcat, w1, b1, w2, b2)


def _phiu_body(vs_ref, vc_ref, es_ref, ec_ref, up_ref, su_ref,
               w1_ref, b1_ref, w2_ref, b2_ref, o_ref):
    va = vs_ref[...] / jnp.maximum(vc_ref[...], 1.0)
    eag = es_ref[...] / jnp.maximum(ec_ref[...], 1.0)
    feat = jnp.concatenate([va, eag, up_ref[...]], axis=1)
    h = _ssp(_mm(feat, w1_ref[...]) + b1_ref[...])
    nu = _ssp(_mm(h, w2_ref[...]) + b2_ref[...])
    o_ref[...] = nu + su_ref[...]


def _phiu(vs, vc, es, ec, up, su, w1, b1, w2, b2):
    return pl.pallas_call(
        _phiu_body,
        out_shape=jax.ShapeDtypeStruct((_B, _EMB), jnp.float32),
    )(vs, vc, es, ec, up, su, w1, b1, w2, b2)


def _q_from_bias(bih_ref, bhh_ref):
    gates = bih_ref[...] + bhh_ref[...]
    i_ = gates[:, 0 * _EMB:1 * _EMB]
    f_ = gates[:, 1 * _EMB:2 * _EMB]
    g_ = gates[:, 2 * _EMB:3 * _EMB]
    o_ = gates[:, 3 * _EMB:4 * _EMB]
    c = jax.nn.sigmoid(i_) * jnp.tanh(g_)
    h = jax.nn.sigmoid(o_) * jnp.tanh(c)
    return h


def _s2s_nodes_body(x_ref, bat_ref, bih_ref, bhh_ref, o_ref):
    q = _q_from_bias(bih_ref, bhh_ref)
    x = x_ref[...]
    e = _rowdot(x, q)
    oh = _onehot(bat_ref[...], _B)
    em = jnp.where(oh > 0.0, e, -1e30)
    m = jnp.max(em, axis=0, keepdims=True)
    m = jnp.where(m < -1e29, 0.0, m)
    mpn = _rowdot(oh, m)
    ex = jnp.exp(e - mpn)
    den = jnp.sum(oh * ex, axis=0, keepdims=True)
    denpn = _rowdot(oh, den)
    a = ex / (denpn + 1e-16)
    r = _mmT0(oh, a * x)
    o_ref[...] = jnp.concatenate(
        [jnp.broadcast_to(q, (_B, _EMB)), r], axis=1)


def _s2s_nodes(x, bat2d, bih, bhh):
    return pl.pallas_call(
        _s2s_nodes_body,
        out_shape=jax.ShapeDtypeStruct((_B, 2 * _EMB), jnp.float32),
    )(x, bat2d, bih, bhh)


def _s2s_edges_body(x_ref, bat_ref, bih_ref, bhh_ref, o_ref,
                    m_s, den_s, rn_s):
    i = pl.program_id(0)
    nsteps = pl.num_programs(0)

    @pl.when(i == 0)
    def _():
        m_s[...] = jnp.full_like(m_s, -1e30)
        den_s[...] = jnp.zeros_like(den_s)
        rn_s[...] = jnp.zeros_like(rn_s)

    q = _q_from_bias(bih_ref, bhh_ref)
    x = x_ref[...]
    e = _rowdot(x, q)
    oh = _onehot(bat_ref[...], _B)
    em = jnp.where(oh > 0.0, e, -1e30)
    mb = jnp.max(em, axis=0, keepdims=True)
    m_old = m_s[...]
    m_new = jnp.maximum(m_old, mb)
    scale = jnp.exp(m_old - m_new)
    mpn = _rowdot(oh, m_new)
    ex = jnp.exp(e - mpn)
    den_b = jnp.sum(oh * ex, axis=0, keepdims=True)
    rn_b = _mmT0(oh, ex * x)
    scale_col = _t_row(scale)
    m_s[...] = m_new
    den_s[...] = den_s[...] * scale + den_b
    rn_s[...] = rn_s[...] * scale_col + rn_b

    @pl.when(i == nsteps - 1)
    def _():
        den_col = _t_row(den_s[...])
        r = rn_s[...] / (den_col + 1e-16)
        o_ref[...] = jnp.concatenate(
            [jnp.broadcast_to(q, (_B, _EMB)), r], axis=1)


def _s2s_edges(x, bat2d, bih, bhh, eblk):
    n = x.shape[0]
    return pl.pallas_call(
        _s2s_edges_body,
        grid=(n // eblk,),
        in_specs=[
            pl.BlockSpec((eblk, _EMB), lambda i: (i, 0)),
            pl.BlockSpec((eblk, 1), lambda i: (i, 0)),
            pl.BlockSpec((1, 4 * _EMB), lambda i: (0, 0)),
            pl.BlockSpec((1, 4 * _EMB), lambda i: (0, 0)),
        ],
        out_specs=pl.BlockSpec((_B, 2 * _EMB), lambda i: (0, 0)),
        out_shape=jax.ShapeDtypeStruct((_B, 2 * _EMB), jnp.float32),
        scratch_shapes=[
            pltpu.VMEM((1, _B), jnp.float32),
            pltpu.VMEM((1, _B), jnp.float32),
            pltpu.VMEM((_B, _EMB), jnp.float32),
        ],
    )(x, bat2d, bih, bhh)


def _head_body(xa_ref, xd_ref, es_ref, u_ref, w1_ref, b1_ref,
               w2_ref, b2_ref, w3_ref, b3_ref, o_ref):
    feat = jnp.concatenate(
        [xa_ref[...], xd_ref[...], es_ref[...], u_ref[...]], axis=1)
    h = _ssp(_mm(feat, w1_ref[...]) + b1_ref[...])
    h = _ssp(_mm(h, w2_ref[...]) + b2_ref[...])
    o_ref[...] = _mm(h, w3_ref[...]) + b3_ref[...]


def _head(xa, xd, es, u, w1, b1, w2, b2, w3, b3):
    return pl.pallas_call(
        _head_body,
        out_shape=jax.ShapeDtypeStruct((_B, 1), jnp.float32),
    )(xa, xd, es, u, w1, b1, w2, b2, w3, b3)


def _stack_lin(plist):
    w = jnp.stack([p["W"] for p in plist])
    b = jnp.stack([p["b"][None, :] for p in plist])
    return w, b


def _layer_weights(p):
    nts = ("atom", "defect")
    ets = ("aa", "ad", "da")
    wn, bn = _stack_lin([p["pre_node"][nt] for nt in nts])
    we, be = _stack_lin([p["pre_edge"][et] for et in ets])
    w1e, b1e = _stack_lin([p["phi_e"][et][0] for et in ets])
    w2e, b2e = _stack_lin([p["phi_e"][et][1] for et in ets])
    w1v, b1v = _stack_lin([p["phi_v"][nt][0] for nt in nts])
    w2v, b2v = _stack_lin([p["phi_v"][nt][1] for nt in nts])
    return dict(wn=wn, bn=bn, we=we, be=be, w1e=w1e, b1e=b1e, w2e=w2e,
                b2e=b2e, w1v=w1v, b1v=b1v, w2v=w2v, b2v=b2v,
                ws=p["pre_state"]["W"], bs=p["pre_state"]["b"][None, :],
                w1u=p["phi_u"][0]["W"], b1u=p["phi_u"][0]["b"][None, :],
                w2u=p["phi_u"][1]["W"], b2u=p["phi_u"][1]["b"][None, :])


def kernel(x_atom, x_defect, edge_index_aa, edge_index_ad, edge_index_da,
           edge_attr_aa, edge_attr_ad, edge_attr_da, state,
           batch_atom, batch_defect, bond_batch_aa, bond_batch_ad,
           bond_batch_da, params):
    n_atom = x_atom.shape[0]
    n_defect = x_defect.shape[0]
    n_aa = edge_index_aa.shape[1]
    n_ad = edge_index_ad.shape[1]
    n_da = edge_index_da.shape[1]
    n_all = n_atom + n_defect
    n_e = n_aa + n_ad + n_da
    eblk = _blk(math.gcd(n_aa, n_ad, n_da), 2000)
    nb_aa, nb_ad = n_aa // eblk, n_ad // eblk

    srccat = jnp.concatenate([
        edge_index_aa[0], edge_index_ad[0], edge_index_da[0] + n_atom])
    dstcat = jnp.concatenate([
        edge_index_aa[1], edge_index_ad[1] + n_atom, edge_index_da[1]])
    bbcat = jnp.concatenate([bond_batch_aa, bond_batch_ad, bond_batch_da])
    bb2d = bbcat[:, None]
    batcat = jnp.concatenate([batch_atom, batch_defect])
    bat2d = batcat[:, None]

    ones_e = jnp.ones((n_e,), jnp.float32)
    cntcat = jax.ops.segment_sum(ones_e, dstcat, num_segments=n_all)[:, None]
    vc = jax.ops.segment_sum(jnp.ones((n_all,), jnp.float32), batcat,
                             num_segments=_B)[:, None]
    ec = jax.ops.segment_sum(ones_e, bbcat, num_segments=_B)[:, None]

    xcat = jnp.concatenate([x_atom, x_defect], axis=0)
    eacat = jnp.concatenate([edge_attr_aa, edge_attr_ad, edge_attr_da], axis=0)
    u = state

    for li, pk in enumerate(("m1", "b1", "b2")):
        w = _layer_weights(params[pk])
        inner = (li == 0)
        xp = _pre_nodes(xcat, w["wn"], w["bn"], n_atom, n_defect)
        up = _lin_small(u, w["ws"], w["bs"])
        gs, gd = _sc_gather(xp, srccat, dstcat)
        zeros_nodes = jnp.zeros((n_all, 128), jnp.float32)
        newe, eanew, es = _edge_layer(
            eacat, gs, gd, bb2d, up, w["we"], w["be"], w["w1e"], w["b1e"],
            w["w2e"], w["b2e"], inner, nb_aa, nb_ad, eblk)
        tot2 = _sc_scatter(newe, dstcat, zeros_nodes, n_all)
        skip = xp if inner else xcat
        xnew, vs = _phiv_layer(xp, tot2, cntcat, bat2d, up, skip,
                               w["w1v"], w["b1v"], w["w2v"], w["b2v"],
                               n_atom, n_defect)
        su = up if inner else u
        u = _phiu(vs, vc, es, ec, up, su,
                  w["w1u"], w["b1u"], w["w2u"], w["b2u"])
        xcat = xnew
        eacat = eanew

    pv = params["sv"]
    xa = _s2s_nodes(xcat[:n_atom], bat2d[:n_atom],
                    (pv["b_ih"] + 0.0)[None, :], pv["b_hh"][None, :])
    pv2 = params["sv2"]
    xd = _s2s_nodes(xcat[n_atom:], bat2d[n_atom:],
                    pv2["b_ih"][None, :], pv2["b_hh"][None, :])
    pe = params["se"]
    es2 = _s2s_edges(eacat, bb2d, pe["b_ih"][None, :], pe["b_hh"][None, :],
                     eblk)
    return _head(xa, xd, es2, u,
                 params["h1"]["W"], params["h1"]["b"][None, :],
                 params["h2"]["W"], params["h2"]["b"][None, :],
                 params["h3"]["W"], params["h3"]["b"][None, :])


# speedup vs baseline: 4.9382x; 1.1224x over previous
import functools
import math

import jax
import jax.numpy as jnp
from jax import lax
from jax.experimental import pallas as pl
from jax.experimental.pallas import tpu as pltpu
from jax.experimental.pallas import tpu_sc as plsc

_LN2 = 0.6931471805599453
_EMB = 32
_B = 64


def _ssp(x):
    m = jnp.maximum(x, 0.0)
    return m + jnp.log(jnp.exp(x - m) + jnp.exp(-m)) - _LN2


def _mm(a, b):
    return jax.lax.dot_general(a, b, (((1,), (0,)), ((), ())),
                               preferred_element_type=jnp.float32)


def _mmT0(a, b):
    return jax.lax.dot_general(a, b, (((0,), (0,)), ((), ())),
                               preferred_element_type=jnp.float32)


def _rowdot(a, v):
    return jnp.sum(a * v, axis=1, keepdims=True)


def _t_row(v):
    eye = (jax.lax.broadcasted_iota(jnp.int32, (_B, _B), 0) ==
           jax.lax.broadcasted_iota(jnp.int32, (_B, _B), 1)).astype(jnp.float32)
    return jnp.sum(eye * v, axis=1, keepdims=True)


def _blk(n, cap):
    for d in range(min(n, cap), 0, -1):
        if n % d == 0:
            return d
    return 1


def _onehot(idx_col, nseg):
    cols = jax.lax.broadcasted_iota(jnp.int32, (idx_col.shape[0], nseg), 1)
    return (idx_col == cols).astype(jnp.float32)


_CHUNK = 128


def _sc_gather(table, srccat, dstcat):
    info = plsc.get_sparse_core_info()
    nw = info.num_cores * info.num_subcores
    n_e = srccat.shape[0]
    nchunks = n_e // _CHUNK
    mesh = plsc.VectorSubcoreMesh(core_axis_name="c", subcore_axis_name="s")

    @functools.partial(
        pl.kernel, mesh=mesh,
        out_type=[jax.ShapeDtypeStruct((n_e, 128), jnp.float32),
                  jax.ShapeDtypeStruct((n_e, 128), jnp.float32)],
        scratch_types=[pltpu.VMEM((_CHUNK,), jnp.int32),
                       pltpu.VMEM((_CHUNK,), jnp.int32),
                       pltpu.VMEM((_CHUNK,), jnp.int32),
                       pltpu.VMEM((_CHUNK,), jnp.int32),
                       pltpu.VMEM((_CHUNK, 128), jnp.float32),
                       pltpu.VMEM((_CHUNK, 128), jnp.float32),
                       pltpu.VMEM((_CHUNK, 128), jnp.float32),
                       pltpu.VMEM((_CHUNK, 128), jnp.float32),
                       pltpu.SemaphoreType.DMA,
                       pltpu.SemaphoreType.DMA,
                       pltpu.SemaphoreType.DMA,
                       pltpu.SemaphoreType.DMA],
    )
    def k(table_h, src_h, dst_h, gs_h, gd_h,
          ixs0, ixs1, ixd0, ixd1, rs0, rs1, rd0, rd1,
          ss0, ss1, sd0, sd1):
        w = lax.axis_index("s") * info.num_cores + lax.axis_index("c")
        c0 = w * nchunks // nw
        c1 = (w + 1) * nchunks // nw
        ixs = (ixs0, ixs1)
        ixd = (ixd0, ixd1)
        rs = (rs0, rs1)
        rd = (rd0, rd1)
        ss = (ss0, ss1)
        sd = (sd0, sd1)

        def start(buf, j):
            b = j * _CHUNK
            pltpu.sync_copy(src_h.at[pl.ds(b, _CHUNK)], ixs[buf])
            pltpu.async_copy(table_h.at[ixs[buf]], rs[buf], ss[buf])
            pltpu.sync_copy(dst_h.at[pl.ds(b, _CHUNK)], ixd[buf])
            pltpu.async_copy(table_h.at[ixd[buf]], rd[buf], sd[buf])

        def finish(buf, j):
            b = j * _CHUNK
            pltpu.make_async_copy(table_h.at[ixs[buf]], rs[buf],
                                  ss[buf]).wait()
            pltpu.sync_copy(rs[buf], gs_h.at[pl.ds(b, _CHUNK)])
            pltpu.make_async_copy(table_h.at[ixd[buf]], rd[buf],
                                  sd[buf]).wait()
            pltpu.sync_copy(rd[buf], gd_h.at[pl.ds(b, _CHUNK)])

        @pl.when(c1 > c0)
        def _():
            start(0, c0)

        def pair(i2, carry):
            for buf in range(2):
                j = c0 + 2 * i2 + buf

                @pl.when(j + 1 < c1)
                def _(buf=buf, j=j):
                    start(1 - buf, j + 1)

                @pl.when(j < c1)
                def _(buf=buf, j=j):
                    finish(buf, j)
            return carry
        lax.fori_loop(0, (c1 - c0 + 1) // 2, pair, 0)

    return k(table, srccat, dstcat)


def _sc_scatter(newe, dstcat, zeros_hbm, n_all):
    info = plsc.get_sparse_core_info()
    nc, ns = info.num_cores, info.num_subcores
    n_e = newe.shape[0]
    nchunks = n_e // _CHUNK
    per_core = nchunks // nc
    nzs = 10
    stripe = n_all // nzs
    mesh = plsc.VectorSubcoreMesh(core_axis_name="c", subcore_axis_name="s")

    @functools.partial(
        pl.kernel, mesh=mesh,
        out_type=jax.ShapeDtypeStruct((nc, n_all, 128), jnp.float32),
        scratch_types=[pltpu.VMEM((_CHUNK,), jnp.int32),
                       pltpu.VMEM((_CHUNK,), jnp.int32),
                       pltpu.VMEM((_CHUNK, 128), jnp.float32),
                       pltpu.VMEM((_CHUNK, 128), jnp.float32),
                       pltpu.VMEM_SHARED((n_all, 128), jnp.float32),
                       pltpu.SemaphoreType.DMA,
                       pltpu.SemaphoreType.DMA],
    )
    def k(ne_h, dst_h, z_h, tot_h, ix0, ix1, r0, r1, acc_sh, s0, s1):
        c = lax.axis_index("c")
        s = lax.axis_index("s")

        @pl.when(s < nzs)
        def _():
            pltpu.sync_copy(z_h.at[pl.ds(s * stripe, stripe)],
                            acc_sh.at[pl.ds(s * stripe, stripe)])
        plsc.subcore_barrier()

        c0 = c * per_core + s * per_core // ns
        c1 = c * per_core + (s + 1) * per_core // ns
        ix = (ix0, ix1)
        rr = (r0, r1)
        sm = (s0, s1)

        def start(buf, j):
            b = j * _CHUNK
            pltpu.sync_copy(dst_h.at[pl.ds(b, _CHUNK)], ix[buf])
            pltpu.async_copy(ne_h.at[pl.ds(b, _CHUNK)], rr[buf], sm[buf])

        def finish(buf, j):
            b = j * _CHUNK
            pltpu.make_async_copy(ne_h.at[pl.ds(b, _CHUNK)], rr[buf],
                                  sm[buf]).wait()
            pltpu.sync_copy(rr[buf], acc_sh.at[ix[buf]], add=True)

        @pl.when(c1 > c0)
        def _():
            start(0, c0)

        def pair(i2, carry):
            for buf in range(2):
                j = c0 + 2 * i2 + buf

                @pl.when(j + 1 < c1)
                def _(buf=buf, j=j):
                    start(1 - buf, j + 1)

                @pl.when(j < c1)
                def _(buf=buf, j=j):
                    finish(buf, j)
            return carry
        lax.fori_loop(0, (c1 - c0 + 1) // 2, pair, 0)
        plsc.subcore_barrier()

        @pl.when(s < nzs)
        def _():
            pltpu.sync_copy(acc_sh.at[pl.ds(s * stripe, stripe)],
                            tot_h.at[c].at[pl.ds(s * stripe, stripe)])

    return k(newe, dstcat, zeros_hbm)


def _pre_nodes_body(x_ref, w_ref, b_ref, o_ref):
    res = _mm(x_ref[...], w_ref[0]) + b_ref[0]
    o_ref[...] = jnp.concatenate(
        [res, jnp.zeros((res.shape[0], 128 - _EMB), jnp.float32)], axis=1)


def _pre_nodes(xcat, w2, b2, n_atom, n_defect):
    din = xcat.shape[1]
    nb = _blk(math.gcd(n_atom, n_defect), 1000)
    nba = n_atom // nb
    grid = (n_atom + n_defect) // nb

    def nt(i):
        return jnp.where(i >= nba, 1, 0)

    return pl.pallas_call(
        _pre_nodes_body,
        grid=(grid,),
        in_specs=[
            pl.BlockSpec((nb, din), lambda i: (i, 0)),
            pl.BlockSpec((1, din, _EMB), lambda i: (nt(i), 0, 0)),
            pl.BlockSpec((1, 1, _EMB), lambda i: (nt(i), 0, 0)),
        ],
        out_specs=pl.BlockSpec((nb, 128), lambda i: (i, 0)),
        out_shape=jax.ShapeDtypeStruct((n_atom + n_defect, 128), jnp.float32),
    )(xcat, w2, b2)


def _lin_body(x_ref, w_ref, b_ref, o_ref):
    o_ref[...] = _mm(x_ref[...], w_ref[...]) + b_ref[...]


def _lin_small(x, w, b2):
    return pl.pallas_call(
        _lin_body,
        out_shape=jax.ShapeDtypeStruct((x.shape[0], w.shape[1]), jnp.float32),
    )(x, w, b2)


def _edge_body(inner_skip, ea_ref, gs_ref, gd_ref, bb_ref, up_ref,
               wpre_ref, bpre_ref, w1_ref, b1_ref, w2_ref, b2_ref,
               ne_ref, eanew_ref, es_ref):
    i = pl.program_id(0)
    ep = _mm(ea_ref[...], wpre_ref[0]) + bpre_ref[0]
    oh = _onehot(bb_ref[...], _B)
    ub = _mm(oh, up_ref[...])
    gs = gs_ref[...][:, :_EMB]
    gd = gd_ref[...][:, :_EMB]
    feat = jnp.concatenate([gs, gd, ep, ub], axis=1)
    h = _ssp(_mm(feat, w1_ref[0]) + b1_ref[0])
    ne = _ssp(_mm(h, w2_ref[0]) + b2_ref[0])
    ne_ref[...] = jnp.concatenate(
        [ne, jnp.zeros((ne.shape[0], 128 - _EMB), jnp.float32)], axis=1)
    skip = ep if inner_skip else ea_ref[...]
    eanew_ref[...] = ne + skip

    @pl.when(i == 0)
    def _():
        es_ref[...] = jnp.zeros_like(es_ref)

    es_ref[...] += _mmT0(oh, ne)


def _edge_layer(eacat, gs, gd, bb2d, up, wpre, bpre, w1, b1, w2, b2,
                inner_skip, nb_aa, nb_ad, eblk):
    n_e = eacat.shape[0]
    din = eacat.shape[1]
    grid = n_e // eblk

    def et(i):
        return jnp.where(i >= nb_aa, 1, 0) + jnp.where(i >= nb_aa + nb_ad, 1, 0)

    return pl.pallas_call(
        functools.partial(_edge_body, inner_skip),
        grid=(grid,),
        in_specs=[
            pl.BlockSpec((eblk, din), lambda i: (i, 0)),
            pl.BlockSpec((eblk, 128), lambda i: (i, 0)),
            pl.BlockSpec((eblk, 128), lambda i: (i, 0)),
            pl.BlockSpec((eblk, 1), lambda i: (i, 0)),
            pl.BlockSpec((_B, _EMB), lambda i: (0, 0)),
            pl.BlockSpec((1, din, _EMB), lambda i: (et(i), 0, 0)),
            pl.BlockSpec((1, 1, _EMB), lambda i: (et(i), 0, 0)),
            pl.BlockSpec((1, 4 * _EMB, 2 * _EMB), lambda i: (et(i), 0, 0)),
            pl.BlockSpec((1, 1, 2 * _EMB), lambda i: (et(i), 0, 0)),
            pl.BlockSpec((1, 2 * _EMB, _EMB), lambda i: (et(i), 0, 0)),
            pl.BlockSpec((1, 1, _EMB), lambda i: (et(i), 0, 0)),
        ],
        out_specs=[
            pl.BlockSpec((eblk, 128), lambda i: (i, 0)),
            pl.BlockSpec((eblk, _EMB), lambda i: (i, 0)),
            pl.BlockSpec((_B, _EMB), lambda i: (0, 0)),
        ],
        out_shape=[
            jax.ShapeDtypeStruct((n_e, 128), jnp.float32),
            jax.ShapeDtypeStruct((n_e, _EMB), jnp.float32),
            jax.ShapeDtypeStruct((_B, _EMB), jnp.float32),
        ],
    )(eacat, gs, gd, bb2d, up, wpre, bpre, w1, b1, w2, b2)


def _phiv_body(xp_ref, tot_ref, cnt_ref, bat_ref, up_ref, skip_ref,
               w1_ref, b1_ref, w2_ref, b2_ref, xn_ref, vs_ref):
    i = pl.program_id(0)
    xp = xp_ref[...][:, :_EMB]
    tot = tot_ref[0][:, :_EMB] + tot_ref[1][:, :_EMB]
    agg = tot / jnp.maximum(cnt_ref[...], 1.0)
    oh = _onehot(bat_ref[...], _B)
    ub = _mm(oh, up_ref[...])
    feat = jnp.concatenate([xp, agg, ub], axis=1)
    h = _ssp(_mm(feat, w1_ref[0]) + b1_ref[0])
    nx = _ssp(_mm(h, w2_ref[0]) + b2_ref[0])
    xn_ref[...] = nx + skip_ref[...][:, :_EMB]

    @pl.when(i == 0)
    def _():
        vs_ref[...] = jnp.zeros_like(vs_ref)

    vs_ref[...] += _mmT0(oh, nx)


def _phiv_layer(xpcat, tot2, cntcat, bat2d, up, skipcat,
                w1, b1, w2, b2, n_atom, n_defect):
    nb = _blk(math.gcd(n_atom, n_defect), 1000)
    nba = n_atom // nb
    n_all = n_atom + n_defect
    grid = n_all // nb

    def nt(i):
        return jnp.where(i >= nba, 1, 0)

    return pl.pallas_call(
        _phiv_body,
        grid=(grid,),
        in_specs=[
            pl.BlockSpec((nb, xpcat.shape[1]), lambda i: (i, 0)),
            pl.BlockSpec((2, nb, tot2.shape[2]), lambda i: (0, i, 0)),
            pl.BlockSpec((nb, 1), lambda i: (i, 0)),
            pl.BlockSpec((nb, 1), lambda i: (i, 0)),
            pl.BlockSpec((_B, _EMB), lambda i: (0, 0)),
            pl.BlockSpec((nb, skipcat.shape[1]), lambda i: (i, 0)),
            pl.BlockSpec((1, 3 * _EMB, 2 * _EMB), lambda i: (nt(i), 0, 0)),
            pl.BlockSpec((1, 1, 2 * _EMB), lambda i: (nt(i), 0, 0)),
            pl.BlockSpec((1, 2 * _EMB, _EMB), lambda i: (nt(i), 0, 0)),
            pl.BlockSpec((1, 1, _EMB), lambda i: (nt(i), 0, 0)),
        ],
        out_specs=[
            pl.BlockSpec((nb, _EMB), lambda i: (i, 0)),
            pl.BlockSpec((_B, _EMB), lambda i: (0, 0)),
        ],
        out_shape=[
            jax.ShapeDtypeStruct((n_all, _EMB), jnp.float32),
            jax.ShapeDtypeStruct((_B, _EMB), jnp.float32),
        ],
    )(xpcat, tot2, cntcat, bat2d, up, skipcat, w1, b1, w2, b2)


def _phiu_body(vs_ref, vc_ref, es_ref, ec_ref, up_ref, su_ref,
               w1_ref, b1_ref, w2_ref, b2_ref, o_ref):
    va = vs_ref[...] / jnp.maximum(vc_ref[...], 1.0)
    eag = es_ref[...] / jnp.maximum(ec_ref[...], 1.0)
    feat = jnp.concatenate([va, eag, up_ref[...]], axis=1)
    h = _ssp(_mm(feat, w1_ref[...]) + b1_ref[...])
    nu = _ssp(_mm(h, w2_ref[...]) + b2_ref[...])
    o_ref[...] = nu + su_ref[...]


def _phiu(vs, vc, es, ec, up, su, w1, b1, w2, b2):
    return pl.pallas_call(
        _phiu_body,
        out_shape=jax.ShapeDtypeStruct((_B, _EMB), jnp.float32),
    )(vs, vc, es, ec, up, su, w1, b1, w2, b2)


def _q_from_bias(bih_ref, bhh_ref):
    gates = bih_ref[...] + bhh_ref[...]
    i_ = gates[:, 0 * _EMB:1 * _EMB]
    f_ = gates[:, 1 * _EMB:2 * _EMB]
    g_ = gates[:, 2 * _EMB:3 * _EMB]
    o_ = gates[:, 3 * _EMB:4 * _EMB]
    c = jax.nn.sigmoid(i_) * jnp.tanh(g_)
    h = jax.nn.sigmoid(o_) * jnp.tanh(c)
    return h


def _s2s_nodes_body(x_ref, bat_ref, bih_ref, bhh_ref, o_ref):
    q = _q_from_bias(bih_ref, bhh_ref)
    x = x_ref[...]
    e = _rowdot(x, q)
    oh = _onehot(bat_ref[...], _B)
    em = jnp.where(oh > 0.0, e, -1e30)
    m = jnp.max(em, axis=0, keepdims=True)
    m = jnp.where(m < -1e29, 0.0, m)
    mpn = _rowdot(oh, m)
    ex = jnp.exp(e - mpn)
    den = jnp.sum(oh * ex, axis=0, keepdims=True)
    denpn = _rowdot(oh, den)
    a = ex / (denpn + 1e-16)
    r = _mmT0(oh, a * x)
    o_ref[...] = jnp.concatenate(
        [jnp.broadcast_to(q, (_B, _EMB)), r], axis=1)


def _s2s_nodes(x, bat2d, bih, bhh):
    return pl.pallas_call(
        _s2s_nodes_body,
        out_shape=jax.ShapeDtypeStruct((_B, 2 * _EMB), jnp.float32),
    )(x, bat2d, bih, bhh)


def _s2s_edges_body(x_ref, bat_ref, bih_ref, bhh_ref, o_ref,
                    m_s, den_s, rn_s):
    i = pl.program_id(0)
    nsteps = pl.num_programs(0)

    @pl.when(i == 0)
    def _():
        m_s[...] = jnp.full_like(m_s, -1e30)
        den_s[...] = jnp.zeros_like(den_s)
        rn_s[...] = jnp.zeros_like(rn_s)

    q = _q_from_bias(bih_ref, bhh_ref)
    x = x_ref[...]
    e = _rowdot(x, q)
    oh = _onehot(bat_ref[...], _B)
    em = jnp.where(oh > 0.0, e, -1e30)
    mb = jnp.max(em, axis=0, keepdims=True)
    m_old = m_s[...]
    m_new = jnp.maximum(m_old, mb)
    scale = jnp.exp(m_old - m_new)
    mpn = _rowdot(oh, m_new)
    ex = jnp.exp(e - mpn)
    den_b = jnp.sum(oh * ex, axis=0, keepdims=True)
    rn_b = _mmT0(oh, ex * x)
    scale_col = _t_row(scale)
    m_s[...] = m_new
    den_s[...] = den_s[...] * scale + den_b
    rn_s[...] = rn_s[...] * scale_col + rn_b

    @pl.when(i == nsteps - 1)
    def _():
        den_col = _t_row(den_s[...])
        r = rn_s[...] / (den_col + 1e-16)
        o_ref[...] = jnp.concatenate(
            [jnp.broadcast_to(q, (_B, _EMB)), r], axis=1)


def _s2s_edges(x, bat2d, bih, bhh, eblk):
    n = x.shape[0]
    return pl.pallas_call(
        _s2s_edges_body,
        grid=(n // eblk,),
        in_specs=[
            pl.BlockSpec((eblk, _EMB), lambda i: (i, 0)),
            pl.BlockSpec((eblk, 1), lambda i: (i, 0)),
            pl.BlockSpec((1, 4 * _EMB), lambda i: (0, 0)),
            pl.BlockSpec((1, 4 * _EMB), lambda i: (0, 0)),
        ],
        out_specs=pl.BlockSpec((_B, 2 * _EMB), lambda i: (0, 0)),
        out_shape=jax.ShapeDtypeStruct((_B, 2 * _EMB), jnp.float32),
        scratch_shapes=[
            pltpu.VMEM((1, _B), jnp.float32),
            pltpu.VMEM((1, _B), jnp.float32),
            pltpu.VMEM((_B, _EMB), jnp.float32),
        ],
    )(x, bat2d, bih, bhh)


def _head_body(xa_ref, xd_ref, es_ref, u_ref, w1_ref, b1_ref,
               w2_ref, b2_ref, w3_ref, b3_ref, o_ref):
    feat = jnp.concatenate(
        [xa_ref[...], xd_ref[...], es_ref[...], u_ref[...]], axis=1)
    h = _ssp(_mm(feat, w1_ref[...]) + b1_ref[...])
    h = _ssp(_mm(h, w2_ref[...]) + b2_ref[...])
    o_ref[...] = _mm(h, w3_ref[...]) + b3_ref[...]


def _head(xa, xd, es, u, w1, b1, w2, b2, w3, b3):
    return pl.pallas_call(
        _head_body,
        out_shape=jax.ShapeDtypeStruct((_B, 1), jnp.float32),
    )(xa, xd, es, u, w1, b1, w2, b2, w3, b3)


def _stack_lin(plist):
    w = jnp.stack([p["W"] for p in plist])
    b = jnp.stack([p["b"][None, :] for p in plist])
    return w, b


def _layer_weights(p):
    nts = ("atom", "defect")
    ets = ("aa", "ad", "da")
    wn, bn = _stack_lin([p["pre_node"][nt] for nt in nts])
    we, be = _stack_lin([p["pre_edge"][et] for et in ets])
    w1e, b1e = _stack_lin([p["phi_e"][et][0] for et in ets])
    w2e, b2e = _stack_lin([p["phi_e"][et][1] for et in ets])
    w1v, b1v = _stack_lin([p["phi_v"][nt][0] for nt in nts])
    w2v, b2v = _stack_lin([p["phi_v"][nt][1] for nt in nts])
    return dict(wn=wn, bn=bn, we=we, be=be, w1e=w1e, b1e=b1e, w2e=w2e,
                b2e=b2e, w1v=w1v, b1v=b1v, w2v=w2v, b2v=b2v,
                ws=p["pre_state"]["W"], bs=p["pre_state"]["b"][None, :],
                w1u=p["phi_u"][0]["W"], b1u=p["phi_u"][0]["b"][None, :],
                w2u=p["phi_u"][1]["W"], b2u=p["phi_u"][1]["b"][None, :])


def kernel(x_atom, x_defect, edge_index_aa, edge_index_ad, edge_index_da,
           edge_attr_aa, edge_attr_ad, edge_attr_da, state,
           batch_atom, batch_defect, bond_batch_aa, bond_batch_ad,
           bond_batch_da, params):
    n_atom = x_atom.shape[0]
    n_defect = x_defect.shape[0]
    n_aa = edge_index_aa.shape[1]
    n_ad = edge_index_ad.shape[1]
    n_da = edge_index_da.shape[1]
    n_all = n_atom + n_defect
    n_e = n_aa + n_ad + n_da
    eblk = _blk(math.gcd(n_aa, n_ad, n_da), 2000)
    nb_aa, nb_ad = n_aa // eblk, n_ad // eblk

    srccat = jnp.concatenate([
        edge_index_aa[0], edge_index_ad[0], edge_index_da[0] + n_atom])
    dstcat = jnp.concatenate([
        edge_index_aa[1], edge_index_ad[1] + n_atom, edge_index_da[1]])
    bbcat = jnp.concatenate([bond_batch_aa, bond_batch_ad, bond_batch_da])
    bb2d = bbcat[:, None]
    batcat = jnp.concatenate([batch_atom, batch_defect])
    bat2d = batcat[:, None]

    ones_e = jnp.ones((n_e,), jnp.float32)
    cntcat = jax.ops.segment_sum(ones_e, dstcat, num_segments=n_all)[:, None]
    vc = jax.ops.segment_sum(jnp.ones((n_all,), jnp.float32), batcat,
                             num_segments=_B)[:, None]
    ec = jax.ops.segment_sum(ones_e, bbcat, num_segments=_B)[:, None]

    xcat = jnp.concatenate([x_atom, x_defect], axis=0)
    eacat = jnp.concatenate([edge_attr_aa, edge_attr_ad, edge_attr_da], axis=0)
    u = state

    for li, pk in enumerate(("m1", "b1", "b2")):
        w = _layer_weights(params[pk])
        inner = (li == 0)
        xp = _pre_nodes(xcat, w["wn"], w["bn"], n_atom, n_defect)
        up = _lin_small(u, w["ws"], w["bs"])
        gs, gd = _sc_gather(xp, srccat, dstcat)
        zeros_nodes = jnp.zeros((n_all, 128), jnp.float32)
        newe, eanew, es = _edge_layer(
            eacat, gs, gd, bb2d, up, w["we"], w["be"], w["w1e"], w["b1e"],
            w["w2e"], w["b2e"], inner, nb_aa, nb_ad, eblk)
        tot2 = _sc_scatter(newe, dstcat, zeros_nodes, n_all)
        skip = xp if inner else xcat
        xnew, vs = _phiv_layer(xp, tot2, cntcat, bat2d, up, skip,
                               w["w1v"], w["b1v"], w["w2v"], w["b2v"],
                               n_atom, n_defect)
        su = up if inner else u
        u = _phiu(vs, vc, es, ec, up, su,
                  w["w1u"], w["b1u"], w["w2u"], w["b2u"])
        xcat = xnew
        eacat = eanew

    pv = params["sv"]
    xa = _s2s_nodes(xcat[:n_atom], bat2d[:n_atom],
                    (pv["b_ih"] + 0.0)[None, :], pv["b_hh"][None, :])
    pv2 = params["sv2"]
    xd = _s2s_nodes(xcat[n_atom:], bat2d[n_atom:],
                    pv2["b_ih"][None, :], pv2["b_hh"][None, :])
    pe = params["se"]
    es2 = _s2s_edges(eacat, bb2d, pe["b_ih"][None, :], pe["b_hh"][None, :],
                     eblk)
    return _head(xa, xd, es2, u,
                 params["h1"]["W"], params["h1"]["b"][None, :],
                 params["h2"]["W"], params["h2"]["b"][None, :],
                 params["h3"]["W"], params["h3"]["b"][None, :])


# speedup vs baseline: 5.2855x; 1.0703x over previous
import functools
import math

import jax
import jax.numpy as jnp
from jax import lax
from jax.experimental import pallas as pl
from jax.experimental.pallas import tpu as pltpu
from jax.experimental.pallas import tpu_sc as plsc

_LN2 = 0.6931471805599453
_EMB = 32
_B = 64


def _ssp(x):
    m = jnp.maximum(x, 0.0)
    return m + jnp.log(jnp.exp(x - m) + jnp.exp(-m)) - _LN2


def _mm(a, b):
    return jax.lax.dot_general(a, b, (((1,), (0,)), ((), ())),
                               preferred_element_type=jnp.float32)


def _mmT0(a, b):
    return jax.lax.dot_general(a, b, (((0,), (0,)), ((), ())),
                               preferred_element_type=jnp.float32)


def _rowdot(a, v):
    return jnp.sum(a * v, axis=1, keepdims=True)


def _t_row(v):
    eye = (jax.lax.broadcasted_iota(jnp.int32, (_B, _B), 0) ==
           jax.lax.broadcasted_iota(jnp.int32, (_B, _B), 1)).astype(jnp.float32)
    return jnp.sum(eye * v, axis=1, keepdims=True)


def _blk(n, cap):
    for d in range(min(n, cap), 0, -1):
        if n % d == 0:
            return d
    return 1


def _onehot(idx_col, nseg):
    cols = jax.lax.broadcasted_iota(jnp.int32, (idx_col.shape[0], nseg), 1)
    return (idx_col == cols).astype(jnp.float32)


_CHUNK = 128


def _sc_gather(table, srccat, dstcat):
    info = plsc.get_sparse_core_info()
    nw = info.num_cores * info.num_subcores
    n_e = srccat.shape[0]
    nchunks = n_e // _CHUNK
    mesh = plsc.VectorSubcoreMesh(core_axis_name="c", subcore_axis_name="s")

    @functools.partial(
        pl.kernel, mesh=mesh,
        out_type=[jax.ShapeDtypeStruct((n_e, 128), jnp.float32),
                  jax.ShapeDtypeStruct((n_e, 128), jnp.float32)],
        scratch_types=[pltpu.VMEM((_CHUNK,), jnp.int32),
                       pltpu.VMEM((_CHUNK,), jnp.int32),
                       pltpu.VMEM((_CHUNK,), jnp.int32),
                       pltpu.VMEM((_CHUNK,), jnp.int32),
                       pltpu.VMEM((_CHUNK, 128), jnp.float32),
                       pltpu.VMEM((_CHUNK, 128), jnp.float32),
                       pltpu.VMEM((_CHUNK, 128), jnp.float32),
                       pltpu.VMEM((_CHUNK, 128), jnp.float32),
                       pltpu.SemaphoreType.DMA,
                       pltpu.SemaphoreType.DMA,
                       pltpu.SemaphoreType.DMA,
                       pltpu.SemaphoreType.DMA],
    )
    def k(table_h, src_h, dst_h, gs_h, gd_h,
          ixs0, ixs1, ixd0, ixd1, rs0, rs1, rd0, rd1,
          ss0, ss1, sd0, sd1):
        w = lax.axis_index("s") * info.num_cores + lax.axis_index("c")
        c0 = w * nchunks // nw
        c1 = (w + 1) * nchunks // nw
        ixs = (ixs0, ixs1)
        ixd = (ixd0, ixd1)
        rs = (rs0, rs1)
        rd = (rd0, rd1)
        ss = (ss0, ss1)
        sd = (sd0, sd1)

        def start(buf, j):
            b = j * _CHUNK
            pltpu.sync_copy(src_h.at[pl.ds(b, _CHUNK)], ixs[buf])
            pltpu.async_copy(table_h.at[ixs[buf]], rs[buf], ss[buf])
            pltpu.sync_copy(dst_h.at[pl.ds(b, _CHUNK)], ixd[buf])
            pltpu.async_copy(table_h.at[ixd[buf]], rd[buf], sd[buf])

        def finish(buf, j):
            b = j * _CHUNK
            pltpu.make_async_copy(table_h.at[ixs[buf]], rs[buf],
                                  ss[buf]).wait()
            pltpu.sync_copy(rs[buf], gs_h.at[pl.ds(b, _CHUNK)])
            pltpu.make_async_copy(table_h.at[ixd[buf]], rd[buf],
                                  sd[buf]).wait()
            pltpu.sync_copy(rd[buf], gd_h.at[pl.ds(b, _CHUNK)])

        @pl.when(c1 > c0)
        def _():
            start(0, c0)

        def pair(i2, carry):
            for buf in range(2):
                j = c0 + 2 * i2 + buf

                @pl.when(j + 1 < c1)
                def _(buf=buf, j=j):
                    start(1 - buf, j + 1)

                @pl.when(j < c1)
                def _(buf=buf, j=j):
                    finish(buf, j)
            return carry
        lax.fori_loop(0, (c1 - c0 + 1) // 2, pair, 0)

    return k(table, srccat, dstcat)


def _sc_scatter(newe, dstcat, zeros_hbm, n_all):
    info = plsc.get_sparse_core_info()
    nc, ns = info.num_cores, info.num_subcores
    n_e = newe.shape[0]
    nchunks = n_e // _CHUNK
    per_core = nchunks // nc
    nzs = 10
    stripe = n_all // nzs
    mesh = plsc.VectorSubcoreMesh(core_axis_name="c", subcore_axis_name="s")

    @functools.partial(
        pl.kernel, mesh=mesh,
        out_type=jax.ShapeDtypeStruct((nc, n_all, 128), jnp.float32),
        scratch_types=[pltpu.VMEM((_CHUNK,), jnp.int32),
                       pltpu.VMEM((_CHUNK,), jnp.int32),
                       pltpu.VMEM((_CHUNK, 128), jnp.float32),
                       pltpu.VMEM((_CHUNK, 128), jnp.float32),
                       pltpu.VMEM_SHARED((n_all, 128), jnp.float32),
                       pltpu.SemaphoreType.DMA,
                       pltpu.SemaphoreType.DMA],
    )
    def k(ne_h, dst_h, z_h, tot_h, ix0, ix1, r0, r1, acc_sh, s0, s1):
        c = lax.axis_index("c")
        s = lax.axis_index("s")

        @pl.when(s < nzs)
        def _():
            pltpu.sync_copy(z_h.at[pl.ds(s * stripe, stripe)],
                            acc_sh.at[pl.ds(s * stripe, stripe)])
        plsc.subcore_barrier()

        c0 = c * per_core + s * per_core // ns
        c1 = c * per_core + (s + 1) * per_core // ns
        ix = (ix0, ix1)
        rr = (r0, r1)
        sm = (s0, s1)

        def start(buf, j):
            b = j * _CHUNK
            pltpu.sync_copy(dst_h.at[pl.ds(b, _CHUNK)], ix[buf])
            pltpu.async_copy(ne_h.at[pl.ds(b, _CHUNK)], rr[buf], sm[buf])

        def finish(buf, j):
            b = j * _CHUNK
            pltpu.make_async_copy(ne_h.at[pl.ds(b, _CHUNK)], rr[buf],
                                  sm[buf]).wait()
            pltpu.sync_copy(rr[buf], acc_sh.at[ix[buf]], add=True)

        @pl.when(c1 > c0)
        def _():
            start(0, c0)

        def pair(i2, carry):
            for buf in range(2):
                j = c0 + 2 * i2 + buf

                @pl.when(j + 1 < c1)
                def _(buf=buf, j=j):
                    start(1 - buf, j + 1)

                @pl.when(j < c1)
                def _(buf=buf, j=j):
                    finish(buf, j)
            return carry
        lax.fori_loop(0, (c1 - c0 + 1) // 2, pair, 0)
        plsc.subcore_barrier()

        @pl.when(s < nzs)
        def _():
            pltpu.sync_copy(acc_sh.at[pl.ds(s * stripe, stripe)],
                            tot_h.at[c].at[pl.ds(s * stripe, stripe)])

    return k(newe, dstcat, zeros_hbm)


def _pre_nodes_body(x_ref, w_ref, b_ref, o_ref):
    res = _mm(x_ref[...], w_ref[0]) + b_ref[0]
    o_ref[...] = jnp.concatenate(
        [res, jnp.zeros((res.shape[0], 128 - _EMB), jnp.float32)], axis=1)


def _pre_nodes(xcat, w2, b2, n_atom, n_defect):
    din = xcat.shape[1]
    nb = _blk(math.gcd(n_atom, n_defect), 2000)
    nba = n_atom // nb
    grid = (n_atom + n_defect) // nb

    def nt(i):
        return jnp.where(i >= nba, 1, 0)

    return pl.pallas_call(
        _pre_nodes_body,
        grid=(grid,),
        in_specs=[
            pl.BlockSpec((nb, din), lambda i: (i, 0)),
            pl.BlockSpec((1, din, _EMB), lambda i: (nt(i), 0, 0)),
            pl.BlockSpec((1, 1, _EMB), lambda i: (nt(i), 0, 0)),
        ],
        out_specs=pl.BlockSpec((nb, 128), lambda i: (i, 0)),
        out_shape=jax.ShapeDtypeStruct((n_atom + n_defect, 128), jnp.float32),
    )(xcat, w2, b2)


def _lin_body(x_ref, w_ref, b_ref, o_ref):
    o_ref[...] = _mm(x_ref[...], w_ref[...]) + b_ref[...]


def _lin_small(x, w, b2):
    return pl.pallas_call(
        _lin_body,
        out_shape=jax.ShapeDtypeStruct((x.shape[0], w.shape[1]), jnp.float32),
    )(x, w, b2)


def _edge_body(inner_skip, ea_ref, gs_ref, gd_ref, bb_ref, up_ref,
               wpre_ref, bpre_ref, w1_ref, b1_ref, w2_ref, b2_ref,
               ne_ref, eanew_ref, es_ref):
    i = pl.program_id(0)
    ep = _mm(ea_ref[...], wpre_ref[0]) + bpre_ref[0]
    oh = _onehot(bb_ref[...], _B)
    ub = _mm(oh, up_ref[...])
    gs = gs_ref[...][:, :_EMB]
    gd = gd_ref[...][:, :_EMB]
    feat = jnp.concatenate([gs, gd, ep, ub], axis=1)
    h = _ssp(_mm(feat, w1_ref[0]) + b1_ref[0])
    ne = _ssp(_mm(h, w2_ref[0]) + b2_ref[0])
    ne_ref[...] = jnp.concatenate(
        [ne, jnp.zeros((ne.shape[0], 128 - _EMB), jnp.float32)], axis=1)
    skip = ep if inner_skip else ea_ref[...]
    eanew_ref[...] = ne + skip

    @pl.when(i == 0)
    def _():
        es_ref[...] = jnp.zeros_like(es_ref)

    es_ref[...] += _mmT0(oh, ne)


def _edge_layer(eacat, gs, gd, bb2d, up, wpre, bpre, w1, b1, w2, b2,
                inner_skip, nb_aa, nb_ad, eblk):
    n_e = eacat.shape[0]
    din = eacat.shape[1]
    grid = n_e // eblk

    def et(i):
        return jnp.where(i >= nb_aa, 1, 0) + jnp.where(i >= nb_aa + nb_ad, 1, 0)

    return pl.pallas_call(
        functools.partial(_edge_body, inner_skip),
        grid=(grid,),
        in_specs=[
            pl.BlockSpec((eblk, din), lambda i: (i, 0)),
            pl.BlockSpec((eblk, 128), lambda i: (i, 0)),
            pl.BlockSpec((eblk, 128), lambda i: (i, 0)),
            pl.BlockSpec((eblk, 1), lambda i: (i, 0)),
            pl.BlockSpec((_B, _EMB), lambda i: (0, 0)),
            pl.BlockSpec((1, din, _EMB), lambda i: (et(i), 0, 0)),
            pl.BlockSpec((1, 1, _EMB), lambda i: (et(i), 0, 0)),
            pl.BlockSpec((1, 4 * _EMB, 2 * _EMB), lambda i: (et(i), 0, 0)),
            pl.BlockSpec((1, 1, 2 * _EMB), lambda i: (et(i), 0, 0)),
            pl.BlockSpec((1, 2 * _EMB, _EMB), lambda i: (et(i), 0, 0)),
            pl.BlockSpec((1, 1, _EMB), lambda i: (et(i), 0, 0)),
        ],
        out_specs=[
            pl.BlockSpec((eblk, 128), lambda i: (i, 0)),
            pl.BlockSpec((eblk, _EMB), lambda i: (i, 0)),
            pl.BlockSpec((_B, _EMB), lambda i: (0, 0)),
        ],
        out_shape=[
            jax.ShapeDtypeStruct((n_e, 128), jnp.float32),
            jax.ShapeDtypeStruct((n_e, _EMB), jnp.float32),
            jax.ShapeDtypeStruct((_B, _EMB), jnp.float32),
        ],
    )(eacat, gs, gd, bb2d, up, wpre, bpre, w1, b1, w2, b2)


def _phiv_body(xp_ref, tot_ref, cnt_ref, bat_ref, up_ref, skip_ref,
               w1_ref, b1_ref, w2_ref, b2_ref, xn_ref, vs_ref):
    i = pl.program_id(0)
    xp = xp_ref[...][:, :_EMB]
    tot = tot_ref[0][:, :_EMB] + tot_ref[1][:, :_EMB]
    agg = tot / jnp.maximum(cnt_ref[...], 1.0)
    oh = _onehot(bat_ref[...], _B)
    ub = _mm(oh, up_ref[...])
    feat = jnp.concatenate([xp, agg, ub], axis=1)
    h = _ssp(_mm(feat, w1_ref[0]) + b1_ref[0])
    nx = _ssp(_mm(h, w2_ref[0]) + b2_ref[0])
    xn_ref[...] = nx + skip_ref[...][:, :_EMB]

    @pl.when(i == 0)
    def _():
        vs_ref[...] = jnp.zeros_like(vs_ref)

    vs_ref[...] += _mmT0(oh, nx)


def _phiv_layer(xpcat, tot2, cntcat, bat2d, up, skipcat,
                w1, b1, w2, b2, n_atom, n_defect):
    nb = _blk(math.gcd(n_atom, n_defect), 2000)
    nba = n_atom // nb
    n_all = n_atom + n_defect
    grid = n_all // nb

    def nt(i):
        return jnp.where(i >= nba, 1, 0)

    return pl.pallas_call(
        _phiv_body,
        grid=(grid,),
        in_specs=[
            pl.BlockSpec((nb, xpcat.shape[1]), lambda i: (i, 0)),
            pl.BlockSpec((2, nb, tot2.shape[2]), lambda i: (0, i, 0)),
            pl.BlockSpec((nb, 1), lambda i: (i, 0)),
            pl.BlockSpec((nb, 1), lambda i: (i, 0)),
            pl.BlockSpec((_B, _EMB), lambda i: (0, 0)),
            pl.BlockSpec((nb, skipcat.shape[1]), lambda i: (i, 0)),
            pl.BlockSpec((1, 3 * _EMB, 2 * _EMB), lambda i: (nt(i), 0, 0)),
            pl.BlockSpec((1, 1, 2 * _EMB), lambda i: (nt(i), 0, 0)),
            pl.BlockSpec((1, 2 * _EMB, _EMB), lambda i: (nt(i), 0, 0)),
            pl.BlockSpec((1, 1, _EMB), lambda i: (nt(i), 0, 0)),
        ],
        out_specs=[
            pl.BlockSpec((nb, _EMB), lambda i: (i, 0)),
            pl.BlockSpec((_B, _EMB), lambda i: (0, 0)),
        ],
        out_shape=[
            jax.ShapeDtypeStruct((n_all, _EMB), jnp.float32),
            jax.ShapeDtypeStruct((_B, _EMB), jnp.float32),
        ],
    )(xpcat, tot2, cntcat, bat2d, up, skipcat, w1, b1, w2, b2)


def _phiu_body(vs_ref, vc_ref, es_ref, ec_ref, up_ref, su_ref,
               w1_ref, b1_ref, w2_ref, b2_ref, o_ref):
    va = vs_ref[...] / jnp.maximum(vc_ref[...], 1.0)
    eag = es_ref[...] / jnp.maximum(ec_ref[...], 1.0)
    feat = jnp.concatenate([va, eag, up_ref[...]], axis=1)
    h = _ssp(_mm(feat, w1_ref[...]) + b1_ref[...])
    nu = _ssp(_mm(h, w2_ref[...]) + b2_ref[...])
    o_ref[...] = nu + su_ref[...]


def _phiu(vs, vc, es, ec, up, su, w1, b1, w2, b2):
    return pl.pallas_call(
        _phiu_body,
        out_shape=jax.ShapeDtypeStruct((_B, _EMB), jnp.float32),
    )(vs, vc, es, ec, up, su, w1, b1, w2, b2)


def _q_from_bias(bih_ref, bhh_ref):
    gates = bih_ref[...] + bhh_ref[...]
    i_ = gates[:, 0 * _EMB:1 * _EMB]
    f_ = gates[:, 1 * _EMB:2 * _EMB]
    g_ = gates[:, 2 * _EMB:3 * _EMB]
    o_ = gates[:, 3 * _EMB:4 * _EMB]
    c = jax.nn.sigmoid(i_) * jnp.tanh(g_)
    h = jax.nn.sigmoid(o_) * jnp.tanh(c)
    return h


def _s2s_nodes_body(x_ref, bat_ref, bih_ref, bhh_ref, o_ref):
    q = _q_from_bias(bih_ref, bhh_ref)
    x = x_ref[...]
    e = _rowdot(x, q)
    oh = _onehot(bat_ref[...], _B)
    em = jnp.where(oh > 0.0, e, -1e30)
    m = jnp.max(em, axis=0, keepdims=True)
    m = jnp.where(m < -1e29, 0.0, m)
    mpn = _rowdot(oh, m)
    ex = jnp.exp(e - mpn)
    den = jnp.sum(oh * ex, axis=0, keepdims=True)
    denpn = _rowdot(oh, den)
    a = ex / (denpn + 1e-16)
    r = _mmT0(oh, a * x)
    o_ref[...] = jnp.concatenate(
        [jnp.broadcast_to(q, (_B, _EMB)), r], axis=1)


def _s2s_nodes(x, bat2d, bih, bhh):
    return pl.pallas_call(
        _s2s_nodes_body,
        out_shape=jax.ShapeDtypeStruct((_B, 2 * _EMB), jnp.float32),
    )(x, bat2d, bih, bhh)


def _s2s_edges_body(x_ref, bat_ref, bih_ref, bhh_ref, o_ref,
                    m_s, den_s, rn_s):
    i = pl.program_id(0)
    nsteps = pl.num_programs(0)

    @pl.when(i == 0)
    def _():
        m_s[...] = jnp.full_like(m_s, -1e30)
        den_s[...] = jnp.zeros_like(den_s)
        rn_s[...] = jnp.zeros_like(rn_s)

    q = _q_from_bias(bih_ref, bhh_ref)
    x = x_ref[...]
    e = _rowdot(x, q)
    oh = _onehot(bat_ref[...], _B)
    em = jnp.where(oh > 0.0, e, -1e30)
    mb = jnp.max(em, axis=0, keepdims=True)
    m_old = m_s[...]
    m_new = jnp.maximum(m_old, mb)
    scale = jnp.exp(m_old - m_new)
    mpn = _rowdot(oh, m_new)
    ex = jnp.exp(e - mpn)
    den_b = jnp.sum(oh * ex, axis=0, keepdims=True)
    rn_b = _mmT0(oh, ex * x)
    scale_col = _t_row(scale)
    m_s[...] = m_new
    den_s[...] = den_s[...] * scale + den_b
    rn_s[...] = rn_s[...] * scale_col + rn_b

    @pl.when(i == nsteps - 1)
    def _():
        den_col = _t_row(den_s[...])
        r = rn_s[...] / (den_col + 1e-16)
        o_ref[...] = jnp.concatenate(
            [jnp.broadcast_to(q, (_B, _EMB)), r], axis=1)


def _s2s_edges(x, bat2d, bih, bhh, eblk):
    n = x.shape[0]
    return pl.pallas_call(
        _s2s_edges_body,
        grid=(n // eblk,),
        in_specs=[
            pl.BlockSpec((eblk, _EMB), lambda i: (i, 0)),
            pl.BlockSpec((eblk, 1), lambda i: (i, 0)),
            pl.BlockSpec((1, 4 * _EMB), lambda i: (0, 0)),
            pl.BlockSpec((1, 4 * _EMB), lambda i: (0, 0)),
        ],
        out_specs=pl.BlockSpec((_B, 2 * _EMB), lambda i: (0, 0)),
        out_shape=jax.ShapeDtypeStruct((_B, 2 * _EMB), jnp.float32),
        scratch_shapes=[
            pltpu.VMEM((1, _B), jnp.float32),
            pltpu.VMEM((1, _B), jnp.float32),
            pltpu.VMEM((_B, _EMB), jnp.float32),
        ],
    )(x, bat2d, bih, bhh)


def _head_body(xa_ref, xd_ref, es_ref, u_ref, w1_ref, b1_ref,
               w2_ref, b2_ref, w3_ref, b3_ref, o_ref):
    feat = jnp.concatenate(
        [xa_ref[...], xd_ref[...], es_ref[...], u_ref[...]], axis=1)
    h = _ssp(_mm(feat, w1_ref[...]) + b1_ref[...])
    h = _ssp(_mm(h, w2_ref[...]) + b2_ref[...])
    o_ref[...] = _mm(h, w3_ref[...]) + b3_ref[...]


def _head(xa, xd, es, u, w1, b1, w2, b2, w3, b3):
    return pl.pallas_call(
        _head_body,
        out_shape=jax.ShapeDtypeStruct((_B, 1), jnp.float32),
    )(xa, xd, es, u, w1, b1, w2, b2, w3, b3)


def _stack_lin(plist):
    w = jnp.stack([p["W"] for p in plist])
    b = jnp.stack([p["b"][None, :] for p in plist])
    return w, b


def _layer_weights(p):
    nts = ("atom", "defect")
    ets = ("aa", "ad", "da")
    wn, bn = _stack_lin([p["pre_node"][nt] for nt in nts])
    we, be = _stack_lin([p["pre_edge"][et] for et in ets])
    w1e, b1e = _stack_lin([p["phi_e"][et][0] for et in ets])
    w2e, b2e = _stack_lin([p["phi_e"][et][1] for et in ets])
    w1v, b1v = _stack_lin([p["phi_v"][nt][0] for nt in nts])
    w2v, b2v = _stack_lin([p["phi_v"][nt][1] for nt in nts])
    return dict(wn=wn, bn=bn, we=we, be=be, w1e=w1e, b1e=b1e, w2e=w2e,
                b2e=b2e, w1v=w1v, b1v=b1v, w2v=w2v, b2v=b2v,
                ws=p["pre_state"]["W"], bs=p["pre_state"]["b"][None, :],
                w1u=p["phi_u"][0]["W"], b1u=p["phi_u"][0]["b"][None, :],
                w2u=p["phi_u"][1]["W"], b2u=p["phi_u"][1]["b"][None, :])


def kernel(x_atom, x_defect, edge_index_aa, edge_index_ad, edge_index_da,
           edge_attr_aa, edge_attr_ad, edge_attr_da, state,
           batch_atom, batch_defect, bond_batch_aa, bond_batch_ad,
           bond_batch_da, params):
    n_atom = x_atom.shape[0]
    n_defect = x_defect.shape[0]
    n_aa = edge_index_aa.shape[1]
    n_ad = edge_index_ad.shape[1]
    n_da = edge_index_da.shape[1]
    n_all = n_atom + n_defect
    n_e = n_aa + n_ad + n_da
    eblk = _blk(math.gcd(n_aa, n_ad, n_da), 4000)
    nb_aa, nb_ad = n_aa // eblk, n_ad // eblk

    srccat = jnp.concatenate([
        edge_index_aa[0], edge_index_ad[0], edge_index_da[0] + n_atom])
    dstcat = jnp.concatenate([
        edge_index_aa[1], edge_index_ad[1] + n_atom, edge_index_da[1]])
    bbcat = jnp.concatenate([bond_batch_aa, bond_batch_ad, bond_batch_da])
    bb2d = bbcat[:, None]
    batcat = jnp.concatenate([batch_atom, batch_defect])
    bat2d = batcat[:, None]

    ones_e = jnp.ones((n_e,), jnp.float32)
    cntcat = jax.ops.segment_sum(ones_e, dstcat, num_segments=n_all)[:, None]
    vc = jax.ops.segment_sum(jnp.ones((n_all,), jnp.float32), batcat,
                             num_segments=_B)[:, None]
    ec = jax.ops.segment_sum(ones_e, bbcat, num_segments=_B)[:, None]

    xcat = jnp.concatenate([x_atom, x_defect], axis=0)
    eacat = jnp.concatenate([edge_attr_aa, edge_attr_ad, edge_attr_da], axis=0)
    u = state

    for li, pk in enumerate(("m1", "b1", "b2")):
        w = _layer_weights(params[pk])
        inner = (li == 0)
        xp = _pre_nodes(xcat, w["wn"], w["bn"], n_atom, n_defect)
        up = _lin_small(u, w["ws"], w["bs"])
        gs, gd = _sc_gather(xp, srccat, dstcat)
        zeros_nodes = jnp.zeros((n_all, 128), jnp.float32)
        newe, eanew, es = _edge_layer(
            eacat, gs, gd, bb2d, up, w["we"], w["be"], w["w1e"], w["b1e"],
            w["w2e"], w["b2e"], inner, nb_aa, nb_ad, eblk)
        tot2 = _sc_scatter(newe, dstcat, zeros_nodes, n_all)
        skip = xp if inner else xcat
        xnew, vs = _phiv_layer(xp, tot2, cntcat, bat2d, up, skip,
                               w["w1v"], w["b1v"], w["w2v"], w["b2v"],
                               n_atom, n_defect)
        su = up if inner else u
        u = _phiu(vs, vc, es, ec, up, su,
                  w["w1u"], w["b1u"], w["w2u"], w["b2u"])
        xcat = xnew
        eacat = eanew

    pv = params["sv"]
    xa = _s2s_nodes(xcat[:n_atom], bat2d[:n_atom],
                    (pv["b_ih"] + 0.0)[None, :], pv["b_hh"][None, :])
    pv2 = params["sv2"]
    xd = _s2s_nodes(xcat[n_atom:], bat2d[n_atom:],
                    pv2["b_ih"][None, :], pv2["b_hh"][None, :])
    pe = params["se"]
    es2 = _s2s_edges(eacat, bb2d, pe["b_ih"][None, :], pe["b_hh"][None, :],
                     eblk)
    return _head(xa, xd, es2, u,
                 params["h1"]["W"], params["h1"]["b"][None, :],
                 params["h2"]["W"], params["h2"]["b"][None, :],
                 params["h3"]["W"], params["h3"]["b"][None, :])


# speedup vs baseline: 5.2858x; 1.0001x over previous
import functools
import math

import jax
import jax.numpy as jnp
from jax import lax
from jax.experimental import pallas as pl
from jax.experimental.pallas import tpu as pltpu
from jax.experimental.pallas import tpu_sc as plsc

_LN2 = 0.6931471805599453
_EMB = 32
_B = 64


def _ssp(x):
    m = jnp.maximum(x, 0.0)
    return m + jnp.log(jnp.exp(x - m) + jnp.exp(-m)) - _LN2


def _mm(a, b):
    return jax.lax.dot_general(a, b, (((1,), (0,)), ((), ())),
                               preferred_element_type=jnp.float32)


def _mmT0(a, b):
    return jax.lax.dot_general(a, b, (((0,), (0,)), ((), ())),
                               preferred_element_type=jnp.float32)


def _rowdot(a, v):
    return jnp.sum(a * v, axis=1, keepdims=True)


def _t_row(v):
    eye = (jax.lax.broadcasted_iota(jnp.int32, (_B, _B), 0) ==
           jax.lax.broadcasted_iota(jnp.int32, (_B, _B), 1)).astype(jnp.float32)
    return jnp.sum(eye * v, axis=1, keepdims=True)


def _blk(n, cap):
    for d in range(min(n, cap), 0, -1):
        if n % d == 0:
            return d
    return 1


def _onehot(idx_col, nseg):
    cols = jax.lax.broadcasted_iota(jnp.int32, (idx_col.shape[0], nseg), 1)
    return (idx_col == cols).astype(jnp.float32)


_CHUNK = 128


def _sc_gather(table, srccat, dstcat):
    info = plsc.get_sparse_core_info()
    nw = info.num_cores * info.num_subcores
    n_e = srccat.shape[0]
    nchunks = n_e // _CHUNK
    mesh = plsc.VectorSubcoreMesh(core_axis_name="c", subcore_axis_name="s")

    @functools.partial(
        pl.kernel, mesh=mesh,
        out_type=[jax.ShapeDtypeStruct((n_e, 128), jnp.float32),
                  jax.ShapeDtypeStruct((n_e, 128), jnp.float32)],
        scratch_types=[pltpu.VMEM((_CHUNK,), jnp.int32),
                       pltpu.VMEM((_CHUNK,), jnp.int32),
                       pltpu.VMEM((_CHUNK,), jnp.int32),
                       pltpu.VMEM((_CHUNK,), jnp.int32),
                       pltpu.VMEM((_CHUNK,), jnp.int32),
                       pltpu.VMEM((_CHUNK,), jnp.int32),
                       pltpu.VMEM((_CHUNK, 128), jnp.float32),
                       pltpu.VMEM((_CHUNK, 128), jnp.float32),
                       pltpu.VMEM((_CHUNK, 128), jnp.float32),
                       pltpu.VMEM((_CHUNK, 128), jnp.float32),
                       pltpu.VMEM((_CHUNK, 128), jnp.float32),
                       pltpu.VMEM((_CHUNK, 128), jnp.float32),
                       pltpu.SemaphoreType.DMA,
                       pltpu.SemaphoreType.DMA,
                       pltpu.SemaphoreType.DMA,
                       pltpu.SemaphoreType.DMA,
                       pltpu.SemaphoreType.DMA,
                       pltpu.SemaphoreType.DMA],
    )
    def k(table_h, src_h, dst_h, gs_h, gd_h,
          ixs0, ixs1, ixs2, ixd0, ixd1, ixd2,
          rs0, rs1, rs2, rd0, rd1, rd2,
          ss0, ss1, ss2, sd0, sd1, sd2):
        w = lax.axis_index("s") * info.num_cores + lax.axis_index("c")
        c0 = w * nchunks // nw
        c1 = (w + 1) * nchunks // nw
        ixs = (ixs0, ixs1, ixs2)
        ixd = (ixd0, ixd1, ixd2)
        rs = (rs0, rs1, rs2)
        rd = (rd0, rd1, rd2)
        ss = (ss0, ss1, ss2)
        sd = (sd0, sd1, sd2)

        def start(buf, j):
            b = j * _CHUNK
            pltpu.sync_copy(src_h.at[pl.ds(b, _CHUNK)], ixs[buf])
            pltpu.async_copy(table_h.at[ixs[buf]], rs[buf], ss[buf])
            pltpu.sync_copy(dst_h.at[pl.ds(b, _CHUNK)], ixd[buf])
            pltpu.async_copy(table_h.at[ixd[buf]], rd[buf], sd[buf])

        def finish(buf, j):
            b = j * _CHUNK
            pltpu.make_async_copy(table_h.at[ixs[buf]], rs[buf],
                                  ss[buf]).wait()
            pltpu.sync_copy(rs[buf], gs_h.at[pl.ds(b, _CHUNK)])
            pltpu.make_async_copy(table_h.at[ixd[buf]], rd[buf],
                                  sd[buf]).wait()
            pltpu.sync_copy(rd[buf], gd_h.at[pl.ds(b, _CHUNK)])

        @pl.when(c1 > c0)
        def _():
            start(0, c0)

        @pl.when(c1 > c0 + 1)
        def _():
            start(1, c0 + 1)

        def trip(i3, carry):
            for buf in range(3):
                j = c0 + 3 * i3 + buf

                @pl.when(j + 2 < c1)
                def _(buf=buf, j=j):
                    start((buf + 2) % 3, j + 2)

                @pl.when(j < c1)
                def _(buf=buf, j=j):
                    finish(buf, j)
            return carry
        lax.fori_loop(0, (c1 - c0 + 2) // 3, trip, 0)

    return k(table, srccat, dstcat)


def _sc_scatter(newe, dstcat, zeros_hbm, n_all):
    info = plsc.get_sparse_core_info()
    nc, ns = info.num_cores, info.num_subcores
    n_e = newe.shape[0]
    nchunks = n_e // _CHUNK
    per_core = nchunks // nc
    nzs = 10
    stripe = n_all // nzs
    mesh = plsc.VectorSubcoreMesh(core_axis_name="c", subcore_axis_name="s")

    @functools.partial(
        pl.kernel, mesh=mesh,
        out_type=jax.ShapeDtypeStruct((nc, n_all, 128), jnp.float32),
        scratch_types=[pltpu.VMEM((_CHUNK,), jnp.int32),
                       pltpu.VMEM((_CHUNK,), jnp.int32),
                       pltpu.VMEM((_CHUNK,), jnp.int32),
                       pltpu.VMEM((_CHUNK, 128), jnp.float32),
                       pltpu.VMEM((_CHUNK, 128), jnp.float32),
                       pltpu.VMEM((_CHUNK, 128), jnp.float32),
                       pltpu.VMEM_SHARED((n_all, 128), jnp.float32),
                       pltpu.SemaphoreType.DMA,
                       pltpu.SemaphoreType.DMA,
                       pltpu.SemaphoreType.DMA],
    )
    def k(ne_h, dst_h, z_h, tot_h, ix0, ix1, ix2, r0, r1, r2, acc_sh,
          s0, s1, s2):
        c = lax.axis_index("c")
        s = lax.axis_index("s")

        @pl.when(s < nzs)
        def _():
            pltpu.sync_copy(z_h.at[pl.ds(s * stripe, stripe)],
                            acc_sh.at[pl.ds(s * stripe, stripe)])
        plsc.subcore_barrier()

        c0 = c * per_core + s * per_core // ns
        c1 = c * per_core + (s + 1) * per_core // ns
        ix = (ix0, ix1, ix2)
        rr = (r0, r1, r2)
        sm = (s0, s1, s2)

        def start(buf, j):
            b = j * _CHUNK
            pltpu.sync_copy(dst_h.at[pl.ds(b, _CHUNK)], ix[buf])
            pltpu.async_copy(ne_h.at[pl.ds(b, _CHUNK)], rr[buf], sm[buf])

        def finish(buf, j):
            b = j * _CHUNK
            pltpu.make_async_copy(ne_h.at[pl.ds(b, _CHUNK)], rr[buf],
                                  sm[buf]).wait()
            pltpu.sync_copy(rr[buf], acc_sh.at[ix[buf]], add=True)

        @pl.when(c1 > c0)
        def _():
            start(0, c0)

        @pl.when(c1 > c0 + 1)
        def _():
            start(1, c0 + 1)

        def trip(i3, carry):
            for buf in range(3):
                j = c0 + 3 * i3 + buf

                @pl.when(j + 2 < c1)
                def _(buf=buf, j=j):
                    start((buf + 2) % 3, j + 2)

                @pl.when(j < c1)
                def _(buf=buf, j=j):
                    finish(buf, j)
            return carry
        lax.fori_loop(0, (c1 - c0 + 2) // 3, trip, 0)
        plsc.subcore_barrier()

        @pl.when(s < nzs)
        def _():
            pltpu.sync_copy(acc_sh.at[pl.ds(s * stripe, stripe)],
                            tot_h.at[c].at[pl.ds(s * stripe, stripe)])

    return k(newe, dstcat, zeros_hbm)


def _pre_nodes_body(x_ref, w_ref, b_ref, o_ref):
    res = _mm(x_ref[...], w_ref[0]) + b_ref[0]
    o_ref[...] = jnp.concatenate(
        [res, jnp.zeros((res.shape[0], 128 - _EMB), jnp.float32)], axis=1)


def _pre_nodes(xcat, w2, b2, n_atom, n_defect):
    din = xcat.shape[1]
    nb = _blk(math.gcd(n_atom, n_defect), 2000)
    nba = n_atom // nb
    grid = (n_atom + n_defect) // nb

    def nt(i):
        return jnp.where(i >= nba, 1, 0)

    return pl.pallas_call(
        _pre_nodes_body,
        grid=(grid,),
        in_specs=[
            pl.BlockSpec((nb, din), lambda i: (i, 0)),
            pl.BlockSpec((1, din, _EMB), lambda i: (nt(i), 0, 0)),
            pl.BlockSpec((1, 1, _EMB), lambda i: (nt(i), 0, 0)),
        ],
        out_specs=pl.BlockSpec((nb, 128), lambda i: (i, 0)),
        out_shape=jax.ShapeDtypeStruct((n_atom + n_defect, 128), jnp.float32),
    )(xcat, w2, b2)


def _lin_body(x_ref, w_ref, b_ref, o_ref):
    o_ref[...] = _mm(x_ref[...], w_ref[...]) + b_ref[...]


def _lin_small(x, w, b2):
    return pl.pallas_call(
        _lin_body,
        out_shape=jax.ShapeDtypeStruct((x.shape[0], w.shape[1]), jnp.float32),
    )(x, w, b2)


def _edge_body(inner_skip, ea_ref, gs_ref, gd_ref, bb_ref, up_ref,
               wpre_ref, bpre_ref, w1_ref, b1_ref, w2_ref, b2_ref,
               ne_ref, eanew_ref, es_ref):
    i = pl.program_id(0)
    ep = _mm(ea_ref[...], wpre_ref[0]) + bpre_ref[0]
    oh = _onehot(bb_ref[...], _B)
    ub = _mm(oh, up_ref[...])
    gs = gs_ref[...][:, :_EMB]
    gd = gd_ref[...][:, :_EMB]
    feat = jnp.concatenate([gs, gd, ep, ub], axis=1)
    h = _ssp(_mm(feat, w1_ref[0]) + b1_ref[0])
    ne = _ssp(_mm(h, w2_ref[0]) + b2_ref[0])
    ne_ref[...] = jnp.concatenate(
        [ne, jnp.zeros((ne.shape[0], 128 - _EMB), jnp.float32)], axis=1)
    skip = ep if inner_skip else ea_ref[...]
    eanew_ref[...] = ne + skip

    @pl.when(i == 0)
    def _():
        es_ref[...] = jnp.zeros_like(es_ref)

    es_ref[...] += _mmT0(oh, ne)


def _edge_layer(eacat, gs, gd, bb2d, up, wpre, bpre, w1, b1, w2, b2,
                inner_skip, nb_aa, nb_ad, eblk):
    n_e = eacat.shape[0]
    din = eacat.shape[1]
    grid = n_e // eblk

    def et(i):
        return jnp.where(i >= nb_aa, 1, 0) + jnp.where(i >= nb_aa + nb_ad, 1, 0)

    return pl.pallas_call(
        functools.partial(_edge_body, inner_skip),
        grid=(grid,),
        in_specs=[
            pl.BlockSpec((eblk, din), lambda i: (i, 0)),
            pl.BlockSpec((eblk, 128), lambda i: (i, 0)),
            pl.BlockSpec((eblk, 128), lambda i: (i, 0)),
            pl.BlockSpec((eblk, 1), lambda i: (i, 0)),
            pl.BlockSpec((_B, _EMB), lambda i: (0, 0)),
            pl.BlockSpec((1, din, _EMB), lambda i: (et(i), 0, 0)),
            pl.BlockSpec((1, 1, _EMB), lambda i: (et(i), 0, 0)),
            pl.BlockSpec((1, 4 * _EMB, 2 * _EMB), lambda i: (et(i), 0, 0)),
            pl.BlockSpec((1, 1, 2 * _EMB), lambda i: (et(i), 0, 0)),
            pl.BlockSpec((1, 2 * _EMB, _EMB), lambda i: (et(i), 0, 0)),
            pl.BlockSpec((1, 1, _EMB), lambda i: (et(i), 0, 0)),
        ],
        out_specs=[
            pl.BlockSpec((eblk, 128), lambda i: (i, 0)),
            pl.BlockSpec((eblk, _EMB), lambda i: (i, 0)),
            pl.BlockSpec((_B, _EMB), lambda i: (0, 0)),
        ],
        out_shape=[
            jax.ShapeDtypeStruct((n_e, 128), jnp.float32),
            jax.ShapeDtypeStruct((n_e, _EMB), jnp.float32),
            jax.ShapeDtypeStruct((_B, _EMB), jnp.float32),
        ],
    )(eacat, gs, gd, bb2d, up, wpre, bpre, w1, b1, w2, b2)


def _phiv_body(xp_ref, tot_ref, cnt_ref, bat_ref, up_ref, skip_ref,
               w1_ref, b1_ref, w2_ref, b2_ref, xn_ref, vs_ref):
    i = pl.program_id(0)
    xp = xp_ref[...][:, :_EMB]
    tot = tot_ref[0][:, :_EMB] + tot_ref[1][:, :_EMB]
    agg = tot / jnp.maximum(cnt_ref[...], 1.0)
    oh = _onehot(bat_ref[...], _B)
    ub = _mm(oh, up_ref[...])
    feat = jnp.concatenate([xp, agg, ub], axis=1)
    h = _ssp(_mm(feat, w1_ref[0]) + b1_ref[0])
    nx = _ssp(_mm(h, w2_ref[0]) + b2_ref[0])
    xn_ref[...] = nx + skip_ref[...][:, :_EMB]

    @pl.when(i == 0)
    def _():
        vs_ref[...] = jnp.zeros_like(vs_ref)

    vs_ref[...] += _mmT0(oh, nx)


def _phiv_layer(xpcat, tot2, cntcat, bat2d, up, skipcat,
                w1, b1, w2, b2, n_atom, n_defect):
    nb = _blk(math.gcd(n_atom, n_defect), 2000)
    nba = n_atom // nb
    n_all = n_atom + n_defect
    grid = n_all // nb

    def nt(i):
        return jnp.where(i >= nba, 1, 0)

    return pl.pallas_call(
        _phiv_body,
        grid=(grid,),
        in_specs=[
            pl.BlockSpec((nb, xpcat.shape[1]), lambda i: (i, 0)),
            pl.BlockSpec((2, nb, tot2.shape[2]), lambda i: (0, i, 0)),
            pl.BlockSpec((nb, 1), lambda i: (i, 0)),
            pl.BlockSpec((nb, 1), lambda i: (i, 0)),
            pl.BlockSpec((_B, _EMB), lambda i: (0, 0)),
            pl.BlockSpec((nb, skipcat.shape[1]), lambda i: (i, 0)),
            pl.BlockSpec((1, 3 * _EMB, 2 * _EMB), lambda i: (nt(i), 0, 0)),
            pl.BlockSpec((1, 1, 2 * _EMB), lambda i: (nt(i), 0, 0)),
            pl.BlockSpec((1, 2 * _EMB, _EMB), lambda i: (nt(i), 0, 0)),
            pl.BlockSpec((1, 1, _EMB), lambda i: (nt(i), 0, 0)),
        ],
        out_specs=[
            pl.BlockSpec((nb, _EMB), lambda i: (i, 0)),
            pl.BlockSpec((_B, _EMB), lambda i: (0, 0)),
        ],
        out_shape=[
            jax.ShapeDtypeStruct((n_all, _EMB), jnp.float32),
            jax.ShapeDtypeStruct((_B, _EMB), jnp.float32),
        ],
    )(xpcat, tot2, cntcat, bat2d, up, skipcat, w1, b1, w2, b2)


def _phiu_body(vs_ref, vc_ref, es_ref, ec_ref, up_ref, su_ref,
               w1_ref, b1_ref, w2_ref, b2_ref, o_ref):
    va = vs_ref[...] / jnp.maximum(vc_ref[...], 1.0)
    eag = es_ref[...] / jnp.maximum(ec_ref[...], 1.0)
    feat = jnp.concatenate([va, eag, up_ref[...]], axis=1)
    h = _ssp(_mm(feat, w1_ref[...]) + b1_ref[...])
    nu = _ssp(_mm(h, w2_ref[...]) + b2_ref[...])
    o_ref[...] = nu + su_ref[...]


def _phiu(vs, vc, es, ec, up, su, w1, b1, w2, b2):
    return pl.pallas_call(
        _phiu_body,
        out_shape=jax.ShapeDtypeStruct((_B, _EMB), jnp.float32),
    )(vs, vc, es, ec, up, su, w1, b1, w2, b2)


def _q_from_bias(bih_ref, bhh_ref):
    gates = bih_ref[...] + bhh_ref[...]
    i_ = gates[:, 0 * _EMB:1 * _EMB]
    f_ = gates[:, 1 * _EMB:2 * _EMB]
    g_ = gates[:, 2 * _EMB:3 * _EMB]
    o_ = gates[:, 3 * _EMB:4 * _EMB]
    c = jax.nn.sigmoid(i_) * jnp.tanh(g_)
    h = jax.nn.sigmoid(o_) * jnp.tanh(c)
    return h


def _s2s_nodes_body(x_ref, bat_ref, bih_ref, bhh_ref, o_ref):
    q = _q_from_bias(bih_ref, bhh_ref)
    x = x_ref[...]
    e = _rowdot(x, q)
    oh = _onehot(bat_ref[...], _B)
    em = jnp.where(oh > 0.0, e, -1e30)
    m = jnp.max(em, axis=0, keepdims=True)
    m = jnp.where(m < -1e29, 0.0, m)
    mpn = _rowdot(oh, m)
    ex = jnp.exp(e - mpn)
    den = jnp.sum(oh * ex, axis=0, keepdims=True)
    denpn = _rowdot(oh, den)
    a = ex / (denpn + 1e-16)
    r = _mmT0(oh, a * x)
    o_ref[...] = jnp.concatenate(
        [jnp.broadcast_to(q, (_B, _EMB)), r], axis=1)


def _s2s_nodes(x, bat2d, bih, bhh):
    return pl.pallas_call(
        _s2s_nodes_body,
        out_shape=jax.ShapeDtypeStruct((_B, 2 * _EMB), jnp.float32),
    )(x, bat2d, bih, bhh)


def _s2s_edges_body(x_ref, bat_ref, bih_ref, bhh_ref, o_ref,
                    m_s, den_s, rn_s):
    i = pl.program_id(0)
    nsteps = pl.num_programs(0)

    @pl.when(i == 0)
    def _():
        m_s[...] = jnp.full_like(m_s, -1e30)
        den_s[...] = jnp.zeros_like(den_s)
        rn_s[...] = jnp.zeros_like(rn_s)

    q = _q_from_bias(bih_ref, bhh_ref)
    x = x_ref[...]
    e = _rowdot(x, q)
    oh = _onehot(bat_ref[...], _B)
    em = jnp.where(oh > 0.0, e, -1e30)
    mb = jnp.max(em, axis=0, keepdims=True)
    m_old = m_s[...]
    m_new = jnp.maximum(m_old, mb)
    scale = jnp.exp(m_old - m_new)
    mpn = _rowdot(oh, m_new)
    ex = jnp.exp(e - mpn)
    den_b = jnp.sum(oh * ex, axis=0, keepdims=True)
    rn_b = _mmT0(oh, ex * x)
    scale_col = _t_row(scale)
    m_s[...] = m_new
    den_s[...] = den_s[...] * scale + den_b
    rn_s[...] = rn_s[...] * scale_col + rn_b

    @pl.when(i == nsteps - 1)
    def _():
        den_col = _t_row(den_s[...])
        r = rn_s[...] / (den_col + 1e-16)
        o_ref[...] = jnp.concatenate(
            [jnp.broadcast_to(q, (_B, _EMB)), r], axis=1)


def _s2s_edges(x, bat2d, bih, bhh, eblk):
    n = x.shape[0]
    return pl.pallas_call(
        _s2s_edges_body,
        grid=(n // eblk,),
        in_specs=[
            pl.BlockSpec((eblk, _EMB), lambda i: (i, 0)),
            pl.BlockSpec((eblk, 1), lambda i: (i, 0)),
            pl.BlockSpec((1, 4 * _EMB), lambda i: (0, 0)),
            pl.BlockSpec((1, 4 * _EMB), lambda i: (0, 0)),
        ],
        out_specs=pl.BlockSpec((_B, 2 * _EMB), lambda i: (0, 0)),
        out_shape=jax.ShapeDtypeStruct((_B, 2 * _EMB), jnp.float32),
        scratch_shapes=[
            pltpu.VMEM((1, _B), jnp.float32),
            pltpu.VMEM((1, _B), jnp.float32),
            pltpu.VMEM((_B, _EMB), jnp.float32),
        ],
    )(x, bat2d, bih, bhh)


def _head_body(xa_ref, xd_ref, es_ref, u_ref, w1_ref, b1_ref,
               w2_ref, b2_ref, w3_ref, b3_ref, o_ref):
    feat = jnp.concatenate(
        [xa_ref[...], xd_ref[...], es_ref[...], u_ref[...]], axis=1)
    h = _ssp(_mm(feat, w1_ref[...]) + b1_ref[...])
    h = _ssp(_mm(h, w2_ref[...]) + b2_ref[...])
    o_ref[...] = _mm(h, w3_ref[...]) + b3_ref[...]


def _head(xa, xd, es, u, w1, b1, w2, b2, w3, b3):
    return pl.pallas_call(
        _head_body,
        out_shape=jax.ShapeDtypeStruct((_B, 1), jnp.float32),
    )(xa, xd, es, u, w1, b1, w2, b2, w3, b3)


def _stack_lin(plist):
    w = jnp.stack([p["W"] for p in plist])
    b = jnp.stack([p["b"][None, :] for p in plist])
    return w, b


def _layer_weights(p):
    nts = ("atom", "defect")
    ets = ("aa", "ad", "da")
    wn, bn = _stack_lin([p["pre_node"][nt] for nt in nts])
    we, be = _stack_lin([p["pre_edge"][et] for et in ets])
    w1e, b1e = _stack_lin([p["phi_e"][et][0] for et in ets])
    w2e, b2e = _stack_lin([p["phi_e"][et][1] for et in ets])
    w1v, b1v = _stack_lin([p["phi_v"][nt][0] for nt in nts])
    w2v, b2v = _stack_lin([p["phi_v"][nt][1] for nt in nts])
    return dict(wn=wn, bn=bn, we=we, be=be, w1e=w1e, b1e=b1e, w2e=w2e,
                b2e=b2e, w1v=w1v, b1v=b1v, w2v=w2v, b2v=b2v,
                ws=p["pre_state"]["W"], bs=p["pre_state"]["b"][None, :],
                w1u=p["phi_u"][0]["W"], b1u=p["phi_u"][0]["b"][None, :],
                w2u=p["phi_u"][1]["W"], b2u=p["phi_u"][1]["b"][None, :])


def kernel(x_atom, x_defect, edge_index_aa, edge_index_ad, edge_index_da,
           edge_attr_aa, edge_attr_ad, edge_attr_da, state,
           batch_atom, batch_defect, bond_batch_aa, bond_batch_ad,
           bond_batch_da, params):
    n_atom = x_atom.shape[0]
    n_defect = x_defect.shape[0]
    n_aa = edge_index_aa.shape[1]
    n_ad = edge_index_ad.shape[1]
    n_da = edge_index_da.shape[1]
    n_all = n_atom + n_defect
    n_e = n_aa + n_ad + n_da
    eblk = _blk(math.gcd(n_aa, n_ad, n_da), 4000)
    nb_aa, nb_ad = n_aa // eblk, n_ad // eblk

    srccat = jnp.concatenate([
        edge_index_aa[0], edge_index_ad[0], edge_index_da[0] + n_atom])
    dstcat = jnp.concatenate([
        edge_index_aa[1], edge_index_ad[1] + n_atom, edge_index_da[1]])
    bbcat = jnp.concatenate([bond_batch_aa, bond_batch_ad, bond_batch_da])
    bb2d = bbcat[:, None]
    batcat = jnp.concatenate([batch_atom, batch_defect])
    bat2d = batcat[:, None]

    ones_e = jnp.ones((n_e,), jnp.float32)
    cntcat = jax.ops.segment_sum(ones_e, dstcat, num_segments=n_all)[:, None]
    vc = jax.ops.segment_sum(jnp.ones((n_all,), jnp.float32), batcat,
                             num_segments=_B)[:, None]
    ec = jax.ops.segment_sum(ones_e, bbcat, num_segments=_B)[:, None]

    xcat = jnp.concatenate([x_atom, x_defect], axis=0)
    eacat = jnp.concatenate([edge_attr_aa, edge_attr_ad, edge_attr_da], axis=0)
    u = state

    for li, pk in enumerate(("m1", "b1", "b2")):
        w = _layer_weights(params[pk])
        inner = (li == 0)
        xp = _pre_nodes(xcat, w["wn"], w["bn"], n_atom, n_defect)
        up = _lin_small(u, w["ws"], w["bs"])
        gs, gd = _sc_gather(xp, srccat, dstcat)
        zeros_nodes = jnp.zeros((n_all, 128), jnp.float32)
        newe, eanew, es = _edge_layer(
            eacat, gs, gd, bb2d, up, w["we"], w["be"], w["w1e"], w["b1e"],
            w["w2e"], w["b2e"], inner, nb_aa, nb_ad, eblk)
        tot2 = _sc_scatter(newe, dstcat, zeros_nodes, n_all)
        skip = xp if inner else xcat
        xnew, vs = _phiv_layer(xp, tot2, cntcat, bat2d, up, skip,
                               w["w1v"], w["b1v"], w["w2v"], w["b2v"],
                               n_atom, n_defect)
        su = up if inner else u
        u = _phiu(vs, vc, es, ec, up, su,
                  w["w1u"], w["b1u"], w["w2u"], w["b2u"])
        xcat = xnew
        eacat = eanew

    pv = params["sv"]
    xa = _s2s_nodes(xcat[:n_atom], bat2d[:n_atom],
                    (pv["b_ih"] + 0.0)[None, :], pv["b_hh"][None, :])
    pv2 = params["sv2"]
    xd = _s2s_nodes(xcat[n_atom:], bat2d[n_atom:],
                    pv2["b_ih"][None, :], pv2["b_hh"][None, :])
    pe = params["se"]
    es2 = _s2s_edges(eacat, bb2d, pe["b_ih"][None, :], pe["b_hh"][None, :],
                     eblk)
    return _head(xa, xd, es2, u,
                 params["h1"]["W"], params["h1"]["b"][None, :],
                 params["h2"]["W"], params["h2"]["b"][None, :],
                 params["h3"]["W"], params["h3"]["b"][None, :])
